# Initial kernel scaffold; baseline (speedup 1.0000x reference)
import jax
import jax.numpy as jnp
from jax.experimental import pallas as pl


def kernel(x, edge_index, edge_attr, batch, W_emb, b_emb, We1, be1, We2, be2, W_root, b_conv, Wih, bih, Whh, bhh, Wih_l, bih_l, Whh_l, bhh_l):
    raise NotImplementedError("write your pallas kernel here")


# speedup vs baseline: 1.3558x; 1.3558x over previous
import functools

import jax
import jax.numpy as jnp
from jax import lax
from jax.experimental import pallas as pl
from jax.experimental.pallas import tpu as pltpu
from jax.experimental.pallas import tpu_sc as plsc

N = 10000
NT = 10240
E = 160000
EP = 163840
D = 16
B = 64
NF = 128

NC = 2
NS = 16
NW = NC * NS
CH = 128
NCH = EP // (NW * CH)
TPT = EP // NW
RPT = NT // NS

BN = 2048
BE = 2048

_f32 = jnp.float32


@functools.lru_cache(None)
def _mesh():
    return plsc.VectorSubcoreMesh(
        core_axis_name="c", subcore_axis_name="s",
        num_cores=NC, num_subcores=NS)


def _prep_body(x_ref, w_ref, b_ref, nf_ref):
    nf_ref[...] = jax.nn.relu(
        jnp.dot(x_ref[...], w_ref[...], preferred_element_type=_f32)
        + b_ref[...])


def _prep(xp, WembT, b_emb2):
    return pl.pallas_call(
        _prep_body,
        grid=(NT // BN,),
        in_specs=[
            pl.BlockSpec((BN, NF), lambda i: (i, 0)),
            pl.BlockSpec((NF, D), lambda i: (0, 0)),
            pl.BlockSpec((1, D), lambda i: (0, 0)),
        ],
        out_specs=pl.BlockSpec((BN, D), lambda i: (i, 0)),
        out_shape=jax.ShapeDtypeStruct((NT, D), _f32),
    )(xp, WembT, b_emb2)


def _edgemlp_body(ea_ref, w1_ref, b1_ref, w2_ref, b2_ref, ew_ref):
    eh = jax.nn.relu(
        jnp.dot(ea_ref[...], w1_ref[...], preferred_element_type=_f32)
        + b1_ref[...])
    ew_ref[...] = (
        jnp.dot(eh, w2_ref[...], preferred_element_type=_f32) + b2_ref[...])


def _edgemlp(eap, We1T, be12, We2T, be22):
    return pl.pallas_call(
        _edgemlp_body,
        grid=(EP // BE,),
        in_specs=[
            pl.BlockSpec((BE, D), lambda i: (i, 0)),
            pl.BlockSpec((D, NF), lambda i: (0, 0)),
            pl.BlockSpec((1, NF), lambda i: (0, 0)),
            pl.BlockSpec((NF, D * D), lambda i: (0, 0)),
            pl.BlockSpec((1, D * D), lambda i: (0, 0)),
        ],
        out_specs=pl.BlockSpec((BE, D * D), lambda i: (i, 0)),
        out_shape=jax.ShapeDtypeStruct((EP, D * D), _f32),
    )(eap, We1T, be12, We2T, be22)


def _msg_body(xj_ref, ew_ref, msg_ref):
    xj = xj_ref[...]
    ew = ew_ref[...]
    acc = xj[:, 0:1] * ew[:, 0:D]
    for i in range(1, D):
        acc = acc + xj[:, i:i + 1] * ew[:, i * D:(i + 1) * D]
    msg_ref[...] = acc


def _msg(xj, eW):
    return pl.pallas_call(
        _msg_body,
        grid=(EP // BE,),
        in_specs=[
            pl.BlockSpec((BE, D), lambda i: (i, 0)),
            pl.BlockSpec((BE, D * D), lambda i: (i, 0)),
        ],
        out_specs=pl.BlockSpec((BE, D), lambda i: (i, 0)),
        out_shape=jax.ShapeDtypeStruct((EP, D), _f32),
    )(xj, eW)


def _gru_body(aggt_ref, degt_ref, st_ref, wr_ref, bc_ref, wih_ref, bih_ref,
              whh_ref, bhh_ref, out_ref):
    aggr = aggt_ref[0] + aggt_ref[1]
    deg = degt_ref[0][:, 0:1] + degt_ref[1][:, 0:1]
    invd = 1.0 / jnp.maximum(deg, 1.0)
    st = st_ref[...]
    conv = jax.nn.relu(
        aggr * invd
        + jnp.dot(st, wr_ref[...], preferred_element_type=_f32)
        + bc_ref[...])
    gi = jnp.dot(conv, wih_ref[...], preferred_element_type=_f32) + bih_ref[...]
    gh = jnp.dot(st, whh_ref[...], preferred_element_type=_f32) + bhh_ref[...]
    r = jax.nn.sigmoid(gi[:, 0:D] + gh[:, 0:D])
    z = jax.nn.sigmoid(gi[:, D:2 * D] + gh[:, D:2 * D])
    n = jnp.tanh(gi[:, 2 * D:3 * D] + r * gh[:, 2 * D:3 * D])
    out_ref[...] = (1.0 - z) * n + z * st


def _gru(aggt, degt, st, WrootT, bc2, WihT, bih2, WhhT, bhh2):
    return pl.pallas_call(
        _gru_body,
        grid=(NT // BN,),
        in_specs=[
            pl.BlockSpec((NC, BN, D), lambda i: (0, i, 0)),
            pl.BlockSpec((NC, BN, D), lambda i: (0, i, 0)),
            pl.BlockSpec((BN, D), lambda i: (i, 0)),
            pl.BlockSpec((D, D), lambda i: (0, 0)),
            pl.BlockSpec((1, D), lambda i: (0, 0)),
            pl.BlockSpec((D, 3 * D), lambda i: (0, 0)),
            pl.BlockSpec((1, 3 * D), lambda i: (0, 0)),
            pl.BlockSpec((D, 3 * D), lambda i: (0, 0)),
            pl.BlockSpec((1, 3 * D), lambda i: (0, 0)),
        ],
        out_specs=pl.BlockSpec((BN, D), lambda i: (i, 0)),
        out_shape=jax.ShapeDtypeStruct((NT, D), _f32),
    )(aggt, degt, st, WrootT, bc2, WihT, bih2, WhhT, bhh2)


def _s2s_body(nv_ref, oh_ref, wih_ref, bih_ref, whh_ref, bhh_ref, out_ref):
    nv = nv_ref[...]
    oh = oh_ref[...]
    q_star = jnp.zeros((B, 2 * D), _f32)
    hL = jnp.zeros((B, D), _f32)
    cL = jnp.zeros((B, D), _f32)
    for _ in range(3):
        gates = (jnp.dot(q_star, wih_ref[...], preferred_element_type=_f32)
                 + bih_ref[...]
                 + jnp.dot(hL, whh_ref[...], preferred_element_type=_f32)
                 + bhh_ref[...])
        i_ = jax.nn.sigmoid(gates[:, 0:D])
        f_ = jax.nn.sigmoid(gates[:, D:2 * D])
        g_ = jnp.tanh(gates[:, 2 * D:3 * D])
        o_ = jax.nn.sigmoid(gates[:, 3 * D:4 * D])
        cL = f_ * cL + i_ * g_
        hL = o_ * jnp.tanh(cL)
        qb = jnp.dot(oh, hL, preferred_element_type=_f32)
        e = jnp.sum(nv * qb, axis=1, keepdims=True)
        m = jnp.where(oh > 0.0, e, -jnp.inf)
        emax = jnp.max(m, axis=0, keepdims=True)
        emax = jnp.where(jnp.isfinite(emax), emax, 0.0)
        enode = jnp.dot(oh, emax.reshape(B, 1),
                        preferred_element_type=_f32)
        a = jnp.exp(e - enode)
        asum = lax.dot_general(oh, a, (((0,), (0,)), ((), ())),
                               preferred_element_type=_f32)
        anode = jnp.dot(oh, asum, preferred_element_type=_f32)
        a = a / (anode + 1e-16)
        r_ = lax.dot_general(oh, a * nv, (((0,), (0,)), ((), ())),
                             preferred_element_type=_f32)
        q_star = jnp.concatenate([hL, r_], axis=1)
    out_ref[...] = q_star


def _s2s(nv, oh, WihlT, bihl2, WhhlT, bhhl2):
    return pl.pallas_call(
        _s2s_body,
        out_shape=jax.ShapeDtypeStruct((B, 2 * D), _f32),
    )(nv, oh, WihlT, bihl2, WhhlT, bhhl2)


def _sc_deg_body(dst_hbm, ones_hbm, zer_hbm, out_hbm, idx_v, obuf, shared, sem):
    c = lax.axis_index("c")
    s = lax.axis_index("s")
    wid = c * NS + s
    pltpu.sync_copy(zer_hbm.at[pl.ds(s * RPT, RPT)],
                    shared.at[pl.ds(s * RPT, RPT)])
    pltpu.sync_copy(ones_hbm, obuf)
    pltpu.sync_copy(dst_hbm.at[wid], idx_v)
    plsc.subcore_barrier()

    def chunk(j, carry):
        pltpu.sync_copy(obuf, shared.at[idx_v.at[j]], add=True)
        return carry

    lax.fori_loop(0, NCH, chunk, 0)
    plsc.subcore_barrier()
    pltpu.sync_copy(shared.at[pl.ds(s * RPT, RPT)],
                    out_hbm.at[c].at[pl.ds(s * RPT, RPT)])


@functools.lru_cache(None)
def _sc_deg_call():
    return functools.partial(
        pl.kernel,
        out_type=jax.ShapeDtypeStruct((NC, NT, D), _f32),
        mesh=_mesh(),
        compiler_params=pltpu.CompilerParams(use_tc_tiling_on_sc=False),
        scratch_types=[
            pltpu.VMEM((NCH, CH), jnp.int32),
            pltpu.VMEM((CH, D), _f32),
            pltpu.VMEM_SHARED((NT, D), _f32),
            pltpu.SemaphoreType.DMA,
        ],
    )(_sc_deg_body)


def _sc_deg(dst, ones, zer):
    return _sc_deg_call()(dst, ones, zer)


def _sc_gather_body(nv_hbm, src_hbm, xj_hbm, idx_v, buf, sem):
    c = lax.axis_index("c")
    s = lax.axis_index("s")
    wid = c * NS + s
    pltpu.sync_copy(src_hbm.at[wid], idx_v)
    base = wid * TPT

    def chunk(j, carry):
        pltpu.async_copy(nv_hbm.at[idx_v.at[j]], buf, sem).wait()
        pltpu.sync_copy(buf, xj_hbm.at[pl.ds(base + j * CH, CH)])
        return carry

    lax.fori_loop(0, NCH, chunk, 0)


@functools.lru_cache(None)
def _sc_gather_call():
    return functools.partial(
        pl.kernel,
        out_type=jax.ShapeDtypeStruct((EP, D), _f32),
        mesh=_mesh(),
        compiler_params=pltpu.CompilerParams(use_tc_tiling_on_sc=False),
        scratch_types=[
            pltpu.VMEM((NCH, CH), jnp.int32),
            pltpu.VMEM((CH, D), _f32),
            pltpu.SemaphoreType.DMA,
        ],
    )(_sc_gather_body)


def _sc_gather(nv, src):
    return _sc_gather_call()(nv, src)


def _sc_scatter_body(msg_hbm, dst_hbm, zer_hbm, out_hbm, idx_v, mbuf, shared,
                     sem):
    c = lax.axis_index("c")
    s = lax.axis_index("s")
    wid = c * NS + s
    pltpu.sync_copy(zer_hbm.at[pl.ds(s * RPT, RPT)],
                    shared.at[pl.ds(s * RPT, RPT)])
    pltpu.sync_copy(dst_hbm.at[wid], idx_v)
    plsc.subcore_barrier()
    base = wid * TPT

    def chunk(j, carry):
        pltpu.sync_copy(msg_hbm.at[pl.ds(base + j * CH, CH)], mbuf)
        pltpu.sync_copy(mbuf, shared.at[idx_v.at[j]], add=True)
        return carry

    lax.fori_loop(0, NCH, chunk, 0)
    plsc.subcore_barrier()
    pltpu.sync_copy(shared.at[pl.ds(s * RPT, RPT)],
                    out_hbm.at[c].at[pl.ds(s * RPT, RPT)])


@functools.lru_cache(None)
def _sc_scatter_call():
    return functools.partial(
        pl.kernel,
        out_type=jax.ShapeDtypeStruct((NC, NT, D), _f32),
        mesh=_mesh(),
        compiler_params=pltpu.CompilerParams(use_tc_tiling_on_sc=False),
        scratch_types=[
            pltpu.VMEM((NCH, CH), jnp.int32),
            pltpu.VMEM((CH, D), _f32),
            pltpu.VMEM_SHARED((NT, D), _f32),
            pltpu.SemaphoreType.DMA,
        ],
    )(_sc_scatter_body)


def _sc_scatter(msg, dst, zer):
    return _sc_scatter_call()(msg, dst, zer)


def kernel(x, edge_index, edge_attr, batch, W_emb, b_emb, We1, be1, We2, be2,
           W_root, b_conv, Wih, bih, Whh, bhh, Wih_l, bih_l, Whh_l, bhh_l):
    xp = jnp.pad(x, ((0, NT - N), (0, 0)))
    src = jnp.pad(edge_index[0], (0, EP - E)).reshape(NW, NCH, CH)
    dst = jnp.pad(edge_index[1], (0, EP - E),
                  constant_values=N).reshape(NW, NCH, CH)
    eap = jnp.pad(edge_attr, ((0, EP - E), (0, 0)))
    batchp = jnp.pad(batch, (0, NT - N), constant_values=B)
    oh = (batchp[:, None] == jnp.arange(B, dtype=batchp.dtype)).astype(_f32)
    zer = jnp.zeros((NT, D), _f32)
    ones = jnp.ones((CH, D), _f32)

    nf = _prep(xp, W_emb.T, b_emb.reshape(1, D))
    eW = _edgemlp(eap, We1.T, be1.reshape(1, NF), We2.T, be2.reshape(1, D * D))
    degt = _sc_deg(dst, ones, zer)

    st = nf
    for _ in range(3):
        xj = _sc_gather(st, src)
        msg = _msg(xj, eW)
        aggt = _sc_scatter(msg, dst, zer)
        st = _gru(aggt, degt, st, W_root.T, b_conv.reshape(1, D),
                  Wih.T, bih.reshape(1, 3 * D), Whh.T, bhh.reshape(1, 3 * D))

    return _s2s(st, oh, Wih_l.T, bih_l.reshape(1, 4 * D),
                Whh_l.T, bhh_l.reshape(1, 4 * D))


# speedup vs baseline: 3.0363x; 2.2394x over previous
import functools

import jax
import jax.numpy as jnp
from jax import lax
from jax.experimental import pallas as pl
from jax.experimental.pallas import tpu as pltpu
from jax.experimental.pallas import tpu_sc as plsc

N = 10000
NT = 10240
E = 160000
EP = 163840
D = 16
B = 64
NF = 128

NC = 2
NS = 16
NW = NC * NS
CH = 128
NCH = EP // (NW * CH)
TPT = EP // NW
RPT = NT // NS

BN = 2048
BE = 2048

_f32 = jnp.float32


@functools.lru_cache(None)
def _mesh():
    return plsc.VectorSubcoreMesh(
        core_axis_name="c", subcore_axis_name="s",
        num_cores=NC, num_subcores=NS)


def _prep_body(x_ref, w_ref, b_ref, nf_ref):
    nf_ref[...] = jax.nn.relu(
        jnp.dot(x_ref[...], w_ref[...], preferred_element_type=_f32)
        + b_ref[...])


def _prep(xp, WembT, b_emb2):
    return pl.pallas_call(
        _prep_body,
        grid=(NT // BN,),
        in_specs=[
            pl.BlockSpec((BN, NF), lambda i: (i, 0)),
            pl.BlockSpec((NF, D), lambda i: (0, 0)),
            pl.BlockSpec((1, D), lambda i: (0, 0)),
        ],
        out_specs=pl.BlockSpec((BN, D), lambda i: (i, 0)),
        out_shape=jax.ShapeDtypeStruct((NT, D), _f32),
    )(xp, WembT, b_emb2)


def _edgemlp_body(ea_ref, w1_ref, b1_ref, w2_ref, b2_ref, ew_ref):
    eh = jax.nn.relu(
        jnp.dot(ea_ref[...], w1_ref[...], preferred_element_type=_f32)
        + b1_ref[...])
    ew_ref[...] = (
        jnp.dot(eh, w2_ref[...], preferred_element_type=_f32) + b2_ref[...])


def _edgemlp(eap, We1T, be12, We2T, be22):
    return pl.pallas_call(
        _edgemlp_body,
        grid=(EP // BE,),
        in_specs=[
            pl.BlockSpec((BE, D), lambda i: (i, 0)),
            pl.BlockSpec((D, NF), lambda i: (0, 0)),
            pl.BlockSpec((1, NF), lambda i: (0, 0)),
            pl.BlockSpec((NF, D * D), lambda i: (0, 0)),
            pl.BlockSpec((1, D * D), lambda i: (0, 0)),
        ],
        out_specs=pl.BlockSpec((BE, D * D), lambda i: (i, 0)),
        out_shape=jax.ShapeDtypeStruct((EP, D * D), _f32),
    )(eap, We1T, be12, We2T, be22)


def _msg_body(xj_ref, ew_ref, s_ref, msg_ref):
    xjr = jnp.tile(xj_ref[...], (1, D))
    msg_ref[...] = jnp.dot(xjr * ew_ref[...], s_ref[...],
                           preferred_element_type=_f32)


def _msg(xj, eW, smat):
    return pl.pallas_call(
        _msg_body,
        grid=(EP // BE,),
        in_specs=[
            pl.BlockSpec((BE, D), lambda i: (i, 0)),
            pl.BlockSpec((BE, D * D), lambda i: (i, 0)),
            pl.BlockSpec((D * D, D), lambda i: (0, 0)),
        ],
        out_specs=pl.BlockSpec((BE, D), lambda i: (i, 0)),
        out_shape=jax.ShapeDtypeStruct((EP, D), _f32),
    )(xj, eW, smat)


def _gru_body(aggt_ref, degt_ref, st_ref, wr_ref, bc_ref, wih_ref, bih_ref,
              whh_ref, bhh_ref, out_ref):
    aggr = aggt_ref[0] + aggt_ref[1]
    deg = degt_ref[0][:, 0:1] + degt_ref[1][:, 0:1]
    invd = 1.0 / jnp.maximum(deg, 1.0)
    st = st_ref[...]
    conv = jax.nn.relu(
        aggr * invd
        + jnp.dot(st, wr_ref[...], preferred_element_type=_f32)
        + bc_ref[...])
    gi = jnp.dot(conv, wih_ref[...], preferred_element_type=_f32) + bih_ref[...]
    gh = jnp.dot(st, whh_ref[...], preferred_element_type=_f32) + bhh_ref[...]
    r = jax.nn.sigmoid(gi[:, 0:D] + gh[:, 0:D])
    z = jax.nn.sigmoid(gi[:, D:2 * D] + gh[:, D:2 * D])
    n = jnp.tanh(gi[:, 2 * D:3 * D] + r * gh[:, 2 * D:3 * D])
    out_ref[...] = (1.0 - z) * n + z * st


def _gru(aggt, degt, st, WrootT, bc2, WihT, bih2, WhhT, bhh2):
    return pl.pallas_call(
        _gru_body,
        grid=(NT // BN,),
        in_specs=[
            pl.BlockSpec((NC, BN, D), lambda i: (0, i, 0)),
            pl.BlockSpec((NC, BN, D), lambda i: (0, i, 0)),
            pl.BlockSpec((BN, D), lambda i: (i, 0)),
            pl.BlockSpec((D, D), lambda i: (0, 0)),
            pl.BlockSpec((1, D), lambda i: (0, 0)),
            pl.BlockSpec((D, 3 * D), lambda i: (0, 0)),
            pl.BlockSpec((1, 3 * D), lambda i: (0, 0)),
            pl.BlockSpec((D, 3 * D), lambda i: (0, 0)),
            pl.BlockSpec((1, 3 * D), lambda i: (0, 0)),
        ],
        out_specs=pl.BlockSpec((BN, D), lambda i: (i, 0)),
        out_shape=jax.ShapeDtypeStruct((NT, D), _f32),
    )(aggt, degt, st, WrootT, bc2, WihT, bih2, WhhT, bhh2)


def _s2s_body(nv_ref, oh_ref, wih_ref, bih_ref, whh_ref, bhh_ref, out_ref):
    nv = nv_ref[...]
    oh = oh_ref[...]
    q_star = jnp.zeros((B, 2 * D), _f32)
    hL = jnp.zeros((B, D), _f32)
    cL = jnp.zeros((B, D), _f32)
    for _ in range(3):
        gates = (jnp.dot(q_star, wih_ref[...], preferred_element_type=_f32)
                 + bih_ref[...]
                 + jnp.dot(hL, whh_ref[...], preferred_element_type=_f32)
                 + bhh_ref[...])
        i_ = jax.nn.sigmoid(gates[:, 0:D])
        f_ = jax.nn.sigmoid(gates[:, D:2 * D])
        g_ = jnp.tanh(gates[:, 2 * D:3 * D])
        o_ = jax.nn.sigmoid(gates[:, 3 * D:4 * D])
        cL = f_ * cL + i_ * g_
        hL = o_ * jnp.tanh(cL)
        qb = jnp.dot(oh, hL, preferred_element_type=_f32)
        e = jnp.sum(nv * qb, axis=1, keepdims=True)
        m = jnp.where(oh > 0.0, e, -jnp.inf)
        emax = jnp.max(m, axis=0, keepdims=True)
        emax = jnp.where(jnp.isfinite(emax), emax, 0.0)
        enode = jnp.dot(oh, emax.reshape(B, 1),
                        preferred_element_type=_f32)
        a = jnp.exp(e - enode)
        asum = lax.dot_general(oh, a, (((0,), (0,)), ((), ())),
                               preferred_element_type=_f32)
        anode = jnp.dot(oh, asum, preferred_element_type=_f32)
        a = a / (anode + 1e-16)
        r_ = lax.dot_general(oh, a * nv, (((0,), (0,)), ((), ())),
                             preferred_element_type=_f32)
        q_star = jnp.concatenate([hL, r_], axis=1)
    out_ref[...] = q_star


def _s2s(nv, oh, WihlT, bihl2, WhhlT, bhhl2):
    return pl.pallas_call(
        _s2s_body,
        out_shape=jax.ShapeDtypeStruct((B, 2 * D), _f32),
    )(nv, oh, WihlT, bihl2, WhhlT, bhhl2)


def _sc_deg_body(dst_hbm, ones_hbm, zer_hbm, out_hbm, idx_v, obuf, shared, sem):
    c = lax.axis_index("c")
    s = lax.axis_index("s")
    wid = c * NS + s
    pltpu.sync_copy(zer_hbm.at[pl.ds(s * RPT, RPT)],
                    shared.at[pl.ds(s * RPT, RPT)])
    pltpu.sync_copy(ones_hbm, obuf)
    pltpu.sync_copy(dst_hbm.at[wid], idx_v)
    plsc.subcore_barrier()

    def fire(k, carry):
        for b in range(8):
            j = k * 8 + b
            pltpu.async_copy(obuf, shared.at[idx_v.at[j]], sem, add=True)
        return carry

    lax.fori_loop(0, NCH // 8, fire, 0)

    def drain(k, carry):
        for b in range(8):
            j = k * 8 + b
            pltpu.make_async_copy(obuf, shared.at[idx_v.at[j]], sem).wait()
        return carry

    lax.fori_loop(0, NCH // 8, drain, 0)
    plsc.subcore_barrier()
    pltpu.sync_copy(shared.at[pl.ds(s * RPT, RPT)],
                    out_hbm.at[c].at[pl.ds(s * RPT, RPT)])


@functools.lru_cache(None)
def _sc_deg_call():
    return functools.partial(
        pl.kernel,
        out_type=jax.ShapeDtypeStruct((NC, NT, D), _f32),
        mesh=_mesh(),
        compiler_params=pltpu.CompilerParams(use_tc_tiling_on_sc=False),
        scratch_types=[
            pltpu.VMEM((NCH, CH), jnp.int32),
            pltpu.VMEM((CH, D), _f32),
            pltpu.VMEM_SHARED((NT, D), _f32),
            pltpu.SemaphoreType.DMA,
        ],
    )(_sc_deg_body)


def _sc_deg(dst, ones, zer):
    return _sc_deg_call()(dst, ones, zer)


def _sc_gather_body(nv_hbm, src_hbm, xj_hbm, idx_v, buf, sem):
    c = lax.axis_index("c")
    s = lax.axis_index("s")
    wid = c * NS + s
    pltpu.sync_copy(src_hbm.at[wid], idx_v)
    base = wid * TPT

    def fire(k, carry):
        for b in range(8):
            j = k * 8 + b
            pltpu.async_copy(nv_hbm.at[idx_v.at[j]],
                             buf.at[pl.ds(j * CH, CH)], sem)
        return carry

    lax.fori_loop(0, NCH // 8, fire, 0)

    def drain(k, carry):
        for b in range(8):
            j = k * 8 + b
            pltpu.make_async_copy(nv_hbm.at[idx_v.at[j]],
                                  buf.at[pl.ds(j * CH, CH)], sem).wait()
        return carry

    lax.fori_loop(0, NCH // 8, drain, 0)
    pltpu.sync_copy(buf, xj_hbm.at[pl.ds(base, TPT)])


@functools.lru_cache(None)
def _sc_gather_call():
    return functools.partial(
        pl.kernel,
        out_type=jax.ShapeDtypeStruct((EP, D), _f32),
        mesh=_mesh(),
        compiler_params=pltpu.CompilerParams(use_tc_tiling_on_sc=False),
        scratch_types=[
            pltpu.VMEM((NCH, CH), jnp.int32),
            pltpu.VMEM((TPT, D), _f32),
            pltpu.SemaphoreType.DMA,
        ],
    )(_sc_gather_body)


def _sc_gather(nv, src):
    return _sc_gather_call()(nv, src)


def _sc_scatter_body(msg_hbm, dst_hbm, zer_hbm, out_hbm, idx_v, mbuf, shared,
                     sem):
    c = lax.axis_index("c")
    s = lax.axis_index("s")
    wid = c * NS + s
    pltpu.sync_copy(zer_hbm.at[pl.ds(s * RPT, RPT)],
                    shared.at[pl.ds(s * RPT, RPT)])
    pltpu.sync_copy(dst_hbm.at[wid], idx_v)
    base = wid * TPT
    pltpu.sync_copy(msg_hbm.at[pl.ds(base, TPT)], mbuf)
    plsc.subcore_barrier()

    def fire(k, carry):
        for b in range(8):
            j = k * 8 + b
            pltpu.async_copy(mbuf.at[pl.ds(j * CH, CH)],
                             shared.at[idx_v.at[j]], sem, add=True)
        return carry

    lax.fori_loop(0, NCH // 8, fire, 0)

    def drain(k, carry):
        for b in range(8):
            j = k * 8 + b
            pltpu.make_async_copy(mbuf.at[pl.ds(j * CH, CH)],
                                  shared.at[idx_v.at[j]], sem).wait()
        return carry

    lax.fori_loop(0, NCH // 8, drain, 0)
    plsc.subcore_barrier()
    pltpu.sync_copy(shared.at[pl.ds(s * RPT, RPT)],
                    out_hbm.at[c].at[pl.ds(s * RPT, RPT)])


@functools.lru_cache(None)
def _sc_scatter_call():
    return functools.partial(
        pl.kernel,
        out_type=jax.ShapeDtypeStruct((NC, NT, D), _f32),
        mesh=_mesh(),
        compiler_params=pltpu.CompilerParams(use_tc_tiling_on_sc=False),
        scratch_types=[
            pltpu.VMEM((NCH, CH), jnp.int32),
            pltpu.VMEM((TPT, D), _f32),
            pltpu.VMEM_SHARED((NT, D), _f32),
            pltpu.SemaphoreType.DMA,
        ],
    )(_sc_scatter_body)


def _sc_scatter(msg, dst, zer):
    return _sc_scatter_call()(msg, dst, zer)


def kernel(x, edge_index, edge_attr, batch, W_emb, b_emb, We1, be1, We2, be2,
           W_root, b_conv, Wih, bih, Whh, bhh, Wih_l, bih_l, Whh_l, bhh_l):
    xp = jnp.pad(x, ((0, NT - N), (0, 0)))
    src = jnp.pad(edge_index[0], (0, EP - E)).reshape(NW, NCH, CH)
    dst = jnp.pad(edge_index[1], (0, EP - E),
                  constant_values=N).reshape(NW, NCH, CH)
    eap = jnp.pad(edge_attr, ((0, EP - E), (0, 0)))
    batchp = jnp.pad(batch, (0, NT - N), constant_values=B)
    oh = (batchp[:, None] == jnp.arange(B, dtype=batchp.dtype)).astype(_f32)
    zer = jnp.zeros((NT, D), _f32)
    ones = jnp.ones((CH, D), _f32)

    jj = jnp.arange(D * D)
    perm = (jj % D) * D + jj // D
    smat = (jj[:, None] // D == jnp.arange(D)[None, :]).astype(_f32)

    nf = _prep(xp, W_emb.T, b_emb.reshape(1, D))
    eW = _edgemlp(eap, We1.T, be1.reshape(1, NF), We2.T[:, perm],
                  be2.reshape(1, D * D)[:, perm])
    degt = _sc_deg(dst, ones, zer)

    st = nf
    for _ in range(3):
        xj = _sc_gather(st, src)
        msg = _msg(xj, eW, smat)
        aggt = _sc_scatter(msg, dst, zer)
        st = _gru(aggt, degt, st, W_root.T, b_conv.reshape(1, D),
                  Wih.T, bih.reshape(1, 3 * D), Whh.T, bhh.reshape(1, 3 * D))

    return _s2s(st, oh, Wih_l.T, bih_l.reshape(1, 4 * D),
                Whh_l.T, bhh_l.reshape(1, 4 * D))


# speedup vs baseline: 3.1482x; 1.0368x over previous
import functools

import jax
import jax.numpy as jnp
from jax import lax
from jax.experimental import pallas as pl
from jax.experimental.pallas import tpu as pltpu
from jax.experimental.pallas import tpu_sc as plsc

N = 10000
NT = 10240
E = 160000
EP = 163840
D = 16
B = 64
NF = 128

NC = 2
NS = 16
NW = NC * NS
CH = 128
NCH = EP // (NW * CH)
TPT = EP // NW
RPT = NT // NS

BN = 2048
BE = 2048

_f32 = jnp.float32


@functools.lru_cache(None)
def _mesh():
    return plsc.VectorSubcoreMesh(
        core_axis_name="c", subcore_axis_name="s",
        num_cores=NC, num_subcores=NS)


def _prep_body(x_ref, w_ref, b_ref, nf_ref):
    nf_ref[...] = jax.nn.relu(
        jnp.dot(x_ref[...], w_ref[...], preferred_element_type=_f32)
        + b_ref[...])


def _prep(xp, WembT, b_emb2):
    return pl.pallas_call(
        _prep_body,
        grid=(NT // BN,),
        in_specs=[
            pl.BlockSpec((BN, NF), lambda i: (i, 0)),
            pl.BlockSpec((NF, D), lambda i: (0, 0)),
            pl.BlockSpec((1, D), lambda i: (0, 0)),
        ],
        out_specs=pl.BlockSpec((BN, D), lambda i: (i, 0)),
        out_shape=jax.ShapeDtypeStruct((NT, D), _f32),
    )(xp, WembT, b_emb2)


def _edgemlp_body(ea_ref, w1_ref, b1_ref, w2_ref, b2_ref, ew_ref):
    eh = jax.nn.relu(
        jnp.dot(ea_ref[...], w1_ref[...], preferred_element_type=_f32)
        + b1_ref[...])
    acc = jnp.dot(eh.astype(jnp.bfloat16), w2_ref[...],
                  preferred_element_type=_f32) + b2_ref[...]
    ew_ref[...] = acc.astype(jnp.bfloat16)


def _edgemlp(eap, We1T, be12, We2T, be22):
    return pl.pallas_call(
        _edgemlp_body,
        grid=(EP // BE,),
        in_specs=[
            pl.BlockSpec((BE, D), lambda i: (i, 0)),
            pl.BlockSpec((D, NF), lambda i: (0, 0)),
            pl.BlockSpec((1, NF), lambda i: (0, 0)),
            pl.BlockSpec((NF, D * D), lambda i: (0, 0)),
            pl.BlockSpec((1, D * D), lambda i: (0, 0)),
        ],
        out_specs=pl.BlockSpec((BE, D * D), lambda i: (i, 0)),
        out_shape=jax.ShapeDtypeStruct((EP, D * D), jnp.bfloat16),
    )(eap, We1T, be12, We2T, be22)


def _msg_body(xj_ref, ew_ref, s_ref, msg_ref):
    xjr = jnp.tile(xj_ref[...], (1, D))
    msg_ref[...] = jnp.dot(xjr * ew_ref[...].astype(_f32), s_ref[...],
                           preferred_element_type=_f32)


def _msg(xj, eW, smat):
    return pl.pallas_call(
        _msg_body,
        grid=(EP // BE,),
        in_specs=[
            pl.BlockSpec((BE, D), lambda i: (i, 0)),
            pl.BlockSpec((BE, D * D), lambda i: (i, 0)),
            pl.BlockSpec((D * D, D), lambda i: (0, 0)),
        ],
        out_specs=pl.BlockSpec((BE, D), lambda i: (i, 0)),
        out_shape=jax.ShapeDtypeStruct((EP, D), _f32),
    )(xj, eW, smat)


def _gru_body(aggt_ref, degt_ref, st_ref, wr_ref, bc_ref, wih_ref, bih_ref,
              whh_ref, bhh_ref, out_ref):
    aggr = aggt_ref[0] + aggt_ref[1]
    deg = degt_ref[0][:, 0:1] + degt_ref[1][:, 0:1]
    invd = 1.0 / jnp.maximum(deg, 1.0)
    st = st_ref[...]
    conv = jax.nn.relu(
        aggr * invd
        + jnp.dot(st, wr_ref[...], preferred_element_type=_f32)
        + bc_ref[...])
    gi = jnp.dot(conv, wih_ref[...], preferred_element_type=_f32) + bih_ref[...]
    gh = jnp.dot(st, whh_ref[...], preferred_element_type=_f32) + bhh_ref[...]
    r = jax.nn.sigmoid(gi[:, 0:D] + gh[:, 0:D])
    z = jax.nn.sigmoid(gi[:, D:2 * D] + gh[:, D:2 * D])
    n = jnp.tanh(gi[:, 2 * D:3 * D] + r * gh[:, 2 * D:3 * D])
    out_ref[...] = (1.0 - z) * n + z * st


def _gru(aggt, degt, st, WrootT, bc2, WihT, bih2, WhhT, bhh2):
    return pl.pallas_call(
        _gru_body,
        grid=(NT // BN,),
        in_specs=[
            pl.BlockSpec((NC, BN, D), lambda i: (0, i, 0)),
            pl.BlockSpec((NC, BN, D), lambda i: (0, i, 0)),
            pl.BlockSpec((BN, D), lambda i: (i, 0)),
            pl.BlockSpec((D, D), lambda i: (0, 0)),
            pl.BlockSpec((1, D), lambda i: (0, 0)),
            pl.BlockSpec((D, 3 * D), lambda i: (0, 0)),
            pl.BlockSpec((1, 3 * D), lambda i: (0, 0)),
            pl.BlockSpec((D, 3 * D), lambda i: (0, 0)),
            pl.BlockSpec((1, 3 * D), lambda i: (0, 0)),
        ],
        out_specs=pl.BlockSpec((BN, D), lambda i: (i, 0)),
        out_shape=jax.ShapeDtypeStruct((NT, D), _f32),
    )(aggt, degt, st, WrootT, bc2, WihT, bih2, WhhT, bhh2)


def _s2s_body(nv_ref, oh_ref, wih_ref, bih_ref, whh_ref, bhh_ref, out_ref):
    nv = nv_ref[...]
    oh = oh_ref[...]
    q_star = jnp.zeros((B, 2 * D), _f32)
    hL = jnp.zeros((B, D), _f32)
    cL = jnp.zeros((B, D), _f32)
    for _ in range(3):
        gates = (jnp.dot(q_star, wih_ref[...], preferred_element_type=_f32)
                 + bih_ref[...]
                 + jnp.dot(hL, whh_ref[...], preferred_element_type=_f32)
                 + bhh_ref[...])
        i_ = jax.nn.sigmoid(gates[:, 0:D])
        f_ = jax.nn.sigmoid(gates[:, D:2 * D])
        g_ = jnp.tanh(gates[:, 2 * D:3 * D])
        o_ = jax.nn.sigmoid(gates[:, 3 * D:4 * D])
        cL = f_ * cL + i_ * g_
        hL = o_ * jnp.tanh(cL)
        qb = jnp.dot(oh, hL, preferred_element_type=_f32)
        e = jnp.sum(nv * qb, axis=1, keepdims=True)
        m = jnp.where(oh > 0.0, e, -jnp.inf)
        emax = jnp.max(m, axis=0, keepdims=True)
        emax = jnp.where(jnp.isfinite(emax), emax, 0.0)
        enode = jnp.dot(oh, emax.reshape(B, 1),
                        preferred_element_type=_f32)
        a = jnp.exp(e - enode)
        asum = lax.dot_general(oh, a, (((0,), (0,)), ((), ())),
                               preferred_element_type=_f32)
        anode = jnp.dot(oh, asum, preferred_element_type=_f32)
        a = a / (anode + 1e-16)
        r_ = lax.dot_general(oh, a * nv, (((0,), (0,)), ((), ())),
                             preferred_element_type=_f32)
        q_star = jnp.concatenate([hL, r_], axis=1)
    out_ref[...] = q_star


def _s2s(nv, oh, WihlT, bihl2, WhhlT, bhhl2):
    return pl.pallas_call(
        _s2s_body,
        out_shape=jax.ShapeDtypeStruct((B, 2 * D), _f32),
    )(nv, oh, WihlT, bihl2, WhhlT, bhhl2)


def _sc_deg_body(dst_hbm, ones_hbm, zer_hbm, out_hbm, idx_v, obuf, shared, sem):
    c = lax.axis_index("c")
    s = lax.axis_index("s")
    wid = c * NS + s
    pltpu.sync_copy(zer_hbm.at[pl.ds(s * RPT, RPT)],
                    shared.at[pl.ds(s * RPT, RPT)])
    pltpu.sync_copy(ones_hbm, obuf)
    pltpu.sync_copy(dst_hbm.at[wid], idx_v)
    plsc.subcore_barrier()

    def fire(k, carry):
        for b in range(8):
            j = k * 8 + b
            pltpu.async_copy(obuf, shared.at[idx_v.at[j]], sem, add=True)
        return carry

    lax.fori_loop(0, NCH // 8, fire, 0)

    def drain(k, carry):
        for b in range(8):
            j = k * 8 + b
            pltpu.make_async_copy(obuf, shared.at[idx_v.at[j]], sem).wait()
        return carry

    lax.fori_loop(0, NCH // 8, drain, 0)
    plsc.subcore_barrier()
    pltpu.sync_copy(shared.at[pl.ds(s * RPT, RPT)],
                    out_hbm.at[c].at[pl.ds(s * RPT, RPT)])


@functools.lru_cache(None)
def _sc_deg_call():
    return functools.partial(
        pl.kernel,
        out_type=jax.ShapeDtypeStruct((NC, NT, D), _f32),
        mesh=_mesh(),
        compiler_params=pltpu.CompilerParams(use_tc_tiling_on_sc=False),
        scratch_types=[
            pltpu.VMEM((NCH, CH), jnp.int32),
            pltpu.VMEM((CH, D), _f32),
            pltpu.VMEM_SHARED((NT, D), _f32),
            pltpu.SemaphoreType.DMA,
        ],
    )(_sc_deg_body)


def _sc_deg(dst, ones, zer):
    return _sc_deg_call()(dst, ones, zer)


def _sc_gather_body(nv_hbm, src_hbm, xj_hbm, idx_v, buf, sem):
    c = lax.axis_index("c")
    s = lax.axis_index("s")
    wid = c * NS + s
    pltpu.sync_copy(src_hbm.at[wid], idx_v)
    base = wid * TPT

    def fire(k, carry):
        for b in range(8):
            j = k * 8 + b
            pltpu.async_copy(nv_hbm.at[idx_v.at[j]],
                             buf.at[pl.ds(j * CH, CH)], sem)
        return carry

    lax.fori_loop(0, NCH // 8, fire, 0)

    def drain(k, carry):
        for b in range(8):
            j = k * 8 + b
            pltpu.make_async_copy(nv_hbm.at[idx_v.at[j]],
                                  buf.at[pl.ds(j * CH, CH)], sem).wait()
        return carry

    lax.fori_loop(0, NCH // 8, drain, 0)
    pltpu.sync_copy(buf, xj_hbm.at[pl.ds(base, TPT)])


@functools.lru_cache(None)
def _sc_gather_call():
    return functools.partial(
        pl.kernel,
        out_type=jax.ShapeDtypeStruct((EP, D), _f32),
        mesh=_mesh(),
        compiler_params=pltpu.CompilerParams(use_tc_tiling_on_sc=False),
        scratch_types=[
            pltpu.VMEM((NCH, CH), jnp.int32),
            pltpu.VMEM((TPT, D), _f32),
            pltpu.SemaphoreType.DMA,
        ],
    )(_sc_gather_body)


def _sc_gather(nv, src):
    return _sc_gather_call()(nv, src)


def _sc_scatter_body(msg_hbm, dst_hbm, zer_hbm, out_hbm, idx_v, mbuf, shared,
                     sem):
    c = lax.axis_index("c")
    s = lax.axis_index("s")
    wid = c * NS + s
    pltpu.sync_copy(zer_hbm.at[pl.ds(s * RPT, RPT)],
                    shared.at[pl.ds(s * RPT, RPT)])
    pltpu.sync_copy(dst_hbm.at[wid], idx_v)
    base = wid * TPT
    pltpu.sync_copy(msg_hbm.at[pl.ds(base, TPT)], mbuf)
    plsc.subcore_barrier()

    def fire(k, carry):
        for b in range(8):
            j = k * 8 + b
            pltpu.async_copy(mbuf.at[pl.ds(j * CH, CH)],
                             shared.at[idx_v.at[j]], sem, add=True)
        return carry

    lax.fori_loop(0, NCH // 8, fire, 0)

    def drain(k, carry):
        for b in range(8):
            j = k * 8 + b
            pltpu.make_async_copy(mbuf.at[pl.ds(j * CH, CH)],
                                  shared.at[idx_v.at[j]], sem).wait()
        return carry

    lax.fori_loop(0, NCH // 8, drain, 0)
    plsc.subcore_barrier()
    pltpu.sync_copy(shared.at[pl.ds(s * RPT, RPT)],
                    out_hbm.at[c].at[pl.ds(s * RPT, RPT)])


@functools.lru_cache(None)
def _sc_scatter_call():
    return functools.partial(
        pl.kernel,
        out_type=jax.ShapeDtypeStruct((NC, NT, D), _f32),
        mesh=_mesh(),
        compiler_params=pltpu.CompilerParams(use_tc_tiling_on_sc=False),
        scratch_types=[
            pltpu.VMEM((NCH, CH), jnp.int32),
            pltpu.VMEM((TPT, D), _f32),
            pltpu.VMEM_SHARED((NT, D), _f32),
            pltpu.SemaphoreType.DMA,
        ],
    )(_sc_scatter_body)


def _sc_scatter(msg, dst, zer):
    return _sc_scatter_call()(msg, dst, zer)


def kernel(x, edge_index, edge_attr, batch, W_emb, b_emb, We1, be1, We2, be2,
           W_root, b_conv, Wih, bih, Whh, bhh, Wih_l, bih_l, Whh_l, bhh_l):
    xp = jnp.pad(x, ((0, NT - N), (0, 0)))
    src = jnp.pad(edge_index[0], (0, EP - E)).reshape(NW, NCH, CH)
    dst = jnp.pad(edge_index[1], (0, EP - E),
                  constant_values=N).reshape(NW, NCH, CH)
    eap = jnp.pad(edge_attr, ((0, EP - E), (0, 0)))
    batchp = jnp.pad(batch, (0, NT - N), constant_values=B)
    oh = (batchp[:, None] == jnp.arange(B, dtype=batchp.dtype)).astype(_f32)
    zer = jnp.zeros((NT, D), _f32)
    ones = jnp.ones((CH, D), _f32)

    jj = jnp.arange(D * D)
    perm = (jj % D) * D + jj // D
    smat = (jj[:, None] // D == jnp.arange(D)[None, :]).astype(_f32)

    nf = _prep(xp, W_emb.T, b_emb.reshape(1, D))
    eW = _edgemlp(eap, We1.T, be1.reshape(1, NF),
                  We2.T[:, perm].astype(jnp.bfloat16),
                  be2.reshape(1, D * D)[:, perm])
    degt = _sc_deg(dst, ones, zer)

    st = nf
    for _ in range(3):
        xj = _sc_gather(st, src)
        msg = _msg(xj, eW, smat)
        aggt = _sc_scatter(msg, dst, zer)
        st = _gru(aggt, degt, st, W_root.T, b_conv.reshape(1, D),
                  Wih.T, bih.reshape(1, 3 * D), Whh.T, bhh.reshape(1, 3 * D))

    return _s2s(st, oh, Wih_l.T, bih_l.reshape(1, 4 * D),
                Whh_l.T, bhh_l.reshape(1, 4 * D))


# speedup vs baseline: 3.5401x; 1.1245x over previous
import functools

import jax
import jax.numpy as jnp
from jax import lax
from jax.experimental import pallas as pl
from jax.experimental.pallas import tpu as pltpu
from jax.experimental.pallas import tpu_sc as plsc

N = 10000
NT = 10240
E = 160000
EP = 163840
D = 16
B = 64
NF = 128

NC = 2
NS = 16
NW = NC * NS
CH = 128
NCH = EP // (NW * CH)
TPT = EP // NW
RPT = NT // NS

BN = 2048
BE = 2048

_f32 = jnp.float32


@functools.lru_cache(None)
def _mesh():
    return plsc.VectorSubcoreMesh(
        core_axis_name="c", subcore_axis_name="s",
        num_cores=NC, num_subcores=NS)


def _prep_body(x_ref, w_ref, b_ref, nf_ref):
    nf_ref[...] = jax.nn.relu(
        jnp.dot(x_ref[...], w_ref[...], preferred_element_type=_f32)
        + b_ref[...])


def _prep(xp, WembT, b_emb2):
    return pl.pallas_call(
        _prep_body,
        grid=(NT // BN,),
        in_specs=[
            pl.BlockSpec((BN, NF), lambda i: (i, 0)),
            pl.BlockSpec((NF, D), lambda i: (0, 0)),
            pl.BlockSpec((1, D), lambda i: (0, 0)),
        ],
        out_specs=pl.BlockSpec((BN, D), lambda i: (i, 0)),
        out_shape=jax.ShapeDtypeStruct((NT, D), _f32),
    )(xp, WembT, b_emb2)


def _edgemlp_body(ea_ref, w1_ref, b1_ref, w2_ref, b2_ref, ew_ref):
    eh = jax.nn.relu(
        jnp.dot(ea_ref[...], w1_ref[...], preferred_element_type=_f32)
        + b1_ref[...])
    acc = jnp.dot(eh.astype(jnp.bfloat16), w2_ref[...],
                  preferred_element_type=_f32) + b2_ref[...]
    ew_ref[...] = acc.astype(jnp.bfloat16)


def _edgemlp(eap, We1T, be12, We2T, be22):
    return pl.pallas_call(
        _edgemlp_body,
        grid=(EP // BE,),
        in_specs=[
            pl.BlockSpec((BE, D), lambda i: (i, 0)),
            pl.BlockSpec((D, NF), lambda i: (0, 0)),
            pl.BlockSpec((1, NF), lambda i: (0, 0)),
            pl.BlockSpec((NF, D * D), lambda i: (0, 0)),
            pl.BlockSpec((1, D * D), lambda i: (0, 0)),
        ],
        out_specs=pl.BlockSpec((BE, D * D), lambda i: (i, 0)),
        out_shape=jax.ShapeDtypeStruct((EP, D * D), jnp.bfloat16),
    )(eap, We1T, be12, We2T, be22)


def _msg_body(xj_ref, ew_ref, r_ref, s_ref, msg_ref):
    xjr = jnp.dot(xj_ref[...], r_ref[...], preferred_element_type=_f32)
    msg_ref[...] = jnp.dot(xjr * ew_ref[...].astype(_f32), s_ref[...],
                           preferred_element_type=_f32)


def _msg(xj, eW, rmat, smat):
    return pl.pallas_call(
        _msg_body,
        grid=(EP // BE,),
        in_specs=[
            pl.BlockSpec((BE, D), lambda i: (i, 0)),
            pl.BlockSpec((BE, D * D), lambda i: (i, 0)),
            pl.BlockSpec((D, D * D), lambda i: (0, 0)),
            pl.BlockSpec((D * D, D), lambda i: (0, 0)),
        ],
        out_specs=pl.BlockSpec((BE, D), lambda i: (i, 0)),
        out_shape=jax.ShapeDtypeStruct((EP, D), _f32),
    )(xj, eW, rmat, smat)


def _gru_body(aggt_ref, degt_ref, st_ref, wr_ref, bc_ref, wih_ref, bih_ref,
              whh_ref, bhh_ref, out_ref):
    aggr = aggt_ref[0] + aggt_ref[1]
    deg = degt_ref[0][:, 0:1] + degt_ref[1][:, 0:1]
    invd = 1.0 / jnp.maximum(deg, 1.0)
    st = st_ref[...]
    conv = jax.nn.relu(
        aggr * invd
        + jnp.dot(st, wr_ref[...], preferred_element_type=_f32)
        + bc_ref[...])
    gi = jnp.dot(conv, wih_ref[...], preferred_element_type=_f32) + bih_ref[...]
    gh = jnp.dot(st, whh_ref[...], preferred_element_type=_f32) + bhh_ref[...]
    r = jax.nn.sigmoid(gi[:, 0:D] + gh[:, 0:D])
    z = jax.nn.sigmoid(gi[:, D:2 * D] + gh[:, D:2 * D])
    n = jnp.tanh(gi[:, 2 * D:3 * D] + r * gh[:, 2 * D:3 * D])
    out_ref[...] = (1.0 - z) * n + z * st


def _gru(aggt, degt, st, WrootT, bc2, WihT, bih2, WhhT, bhh2):
    return pl.pallas_call(
        _gru_body,
        grid=(NT // BN,),
        in_specs=[
            pl.BlockSpec((NC, BN, D), lambda i: (0, i, 0)),
            pl.BlockSpec((NC, BN, D), lambda i: (0, i, 0)),
            pl.BlockSpec((BN, D), lambda i: (i, 0)),
            pl.BlockSpec((D, D), lambda i: (0, 0)),
            pl.BlockSpec((1, D), lambda i: (0, 0)),
            pl.BlockSpec((D, 3 * D), lambda i: (0, 0)),
            pl.BlockSpec((1, 3 * D), lambda i: (0, 0)),
            pl.BlockSpec((D, 3 * D), lambda i: (0, 0)),
            pl.BlockSpec((1, 3 * D), lambda i: (0, 0)),
        ],
        out_specs=pl.BlockSpec((BN, D), lambda i: (i, 0)),
        out_shape=jax.ShapeDtypeStruct((NT, D), _f32),
    )(aggt, degt, st, WrootT, bc2, WihT, bih2, WhhT, bhh2)


def _s2s_body(nv_ref, oh_ref, wih_ref, bih_ref, whh_ref, bhh_ref, out_ref):
    nv = nv_ref[...]
    oh = oh_ref[...]
    q_star = jnp.zeros((B, 2 * D), _f32)
    hL = jnp.zeros((B, D), _f32)
    cL = jnp.zeros((B, D), _f32)
    for _ in range(3):
        gates = (jnp.dot(q_star, wih_ref[...], preferred_element_type=_f32)
                 + bih_ref[...]
                 + jnp.dot(hL, whh_ref[...], preferred_element_type=_f32)
                 + bhh_ref[...])
        i_ = jax.nn.sigmoid(gates[:, 0:D])
        f_ = jax.nn.sigmoid(gates[:, D:2 * D])
        g_ = jnp.tanh(gates[:, 2 * D:3 * D])
        o_ = jax.nn.sigmoid(gates[:, 3 * D:4 * D])
        cL = f_ * cL + i_ * g_
        hL = o_ * jnp.tanh(cL)
        qb = jnp.dot(oh, hL, preferred_element_type=_f32)
        e = jnp.sum(nv * qb, axis=1, keepdims=True)
        m = jnp.where(oh > 0.0, e, -jnp.inf)
        emax = jnp.max(m, axis=0, keepdims=True)
        emax = jnp.where(jnp.isfinite(emax), emax, 0.0)
        enode = jnp.dot(oh, emax.reshape(B, 1),
                        preferred_element_type=_f32)
        a = jnp.exp(e - enode)
        asum = lax.dot_general(oh, a, (((0,), (0,)), ((), ())),
                               preferred_element_type=_f32)
        anode = jnp.dot(oh, asum, preferred_element_type=_f32)
        a = a / (anode + 1e-16)
        r_ = lax.dot_general(oh, a * nv, (((0,), (0,)), ((), ())),
                             preferred_element_type=_f32)
        q_star = jnp.concatenate([hL, r_], axis=1)
    out_ref[...] = q_star


def _s2s(nv, oh, WihlT, bihl2, WhhlT, bhhl2):
    return pl.pallas_call(
        _s2s_body,
        out_shape=jax.ShapeDtypeStruct((B, 2 * D), _f32),
    )(nv, oh, WihlT, bihl2, WhhlT, bhhl2)


def _sc_gather_body(nv_hbm, src_hbm, xj_hbm, idx_v, buf, sem):
    c = lax.axis_index("c")
    s = lax.axis_index("s")
    wid = c * NS + s
    pltpu.sync_copy(src_hbm.at[wid], idx_v)
    base = wid * TPT

    def fire(k, carry):
        for b in range(8):
            j = k * 8 + b
            pltpu.async_copy(nv_hbm.at[idx_v.at[j]],
                             buf.at[pl.ds(j * CH, CH)], sem)
        return carry

    lax.fori_loop(0, NCH // 8, fire, 0)

    def drain(k, carry):
        for b in range(8):
            j = k * 8 + b
            pltpu.make_async_copy(nv_hbm.at[idx_v.at[j]],
                                  buf.at[pl.ds(j * CH, CH)], sem).wait()
        return carry

    lax.fori_loop(0, NCH // 8, drain, 0)
    pltpu.sync_copy(buf, xj_hbm.at[pl.ds(base, TPT)])


@functools.lru_cache(None)
def _sc_gather_call():
    return functools.partial(
        pl.kernel,
        out_type=jax.ShapeDtypeStruct((EP, D), _f32),
        mesh=_mesh(),
        compiler_params=pltpu.CompilerParams(use_tc_tiling_on_sc=False),
        scratch_types=[
            pltpu.VMEM((NCH, CH), jnp.int32),
            pltpu.VMEM((TPT, D), _f32),
            pltpu.SemaphoreType.DMA,
        ],
    )(_sc_gather_body)


def _sc_gather(nv, src):
    return _sc_gather_call()(nv, src)


def _make_sc_scatter_body(with_deg):
    def body(*args):
        if with_deg:
            (msg_hbm, dst_hbm, zer_hbm, ones_hbm, out_hbm, deg_hbm,
             idx_v, mbuf, obuf, shared, shared_deg, sem) = args
        else:
            (msg_hbm, dst_hbm, zer_hbm, out_hbm,
             idx_v, mbuf, shared, sem) = args
        c = lax.axis_index("c")
        s = lax.axis_index("s")
        wid = c * NS + s
        pltpu.sync_copy(zer_hbm.at[pl.ds(s * RPT, RPT)],
                        shared.at[pl.ds(s * RPT, RPT)])
        if with_deg:
            pltpu.sync_copy(zer_hbm.at[pl.ds(s * RPT, RPT)],
                            shared_deg.at[pl.ds(s * RPT, RPT)])
            pltpu.sync_copy(ones_hbm, obuf)
        pltpu.sync_copy(dst_hbm.at[wid], idx_v)
        base = wid * TPT
        pltpu.sync_copy(msg_hbm.at[pl.ds(base, TPT)], mbuf)
        plsc.subcore_barrier()

        def fire(k, carry):
            for b in range(8):
                j = k * 8 + b
                pltpu.async_copy(mbuf.at[pl.ds(j * CH, CH)],
                                 shared.at[idx_v.at[j]], sem, add=True)
                if with_deg:
                    pltpu.async_copy(obuf, shared_deg.at[idx_v.at[j]], sem,
                                     add=True)
            return carry

        lax.fori_loop(0, NCH // 8, fire, 0)

        def drain(k, carry):
            for b in range(8):
                j = k * 8 + b
                pltpu.make_async_copy(mbuf.at[pl.ds(j * CH, CH)],
                                      shared.at[idx_v.at[j]], sem).wait()
                if with_deg:
                    pltpu.make_async_copy(obuf, shared_deg.at[idx_v.at[j]],
                                          sem).wait()
            return carry

        lax.fori_loop(0, NCH // 8, drain, 0)
        plsc.subcore_barrier()
        pltpu.sync_copy(shared.at[pl.ds(s * RPT, RPT)],
                        out_hbm.at[c].at[pl.ds(s * RPT, RPT)])
        if with_deg:
            pltpu.sync_copy(shared_deg.at[pl.ds(s * RPT, RPT)],
                            deg_hbm.at[c].at[pl.ds(s * RPT, RPT)])

    return body


@functools.lru_cache(None)
def _sc_scatter_call(with_deg):
    out_type = jax.ShapeDtypeStruct((NC, NT, D), _f32)
    scratch = [
        pltpu.VMEM((NCH, CH), jnp.int32),
        pltpu.VMEM((TPT, D), _f32),
    ]
    if with_deg:
        out_type = (out_type, jax.ShapeDtypeStruct((NC, NT, D), _f32))
        scratch.append(pltpu.VMEM((CH, D), _f32))
    scratch.append(pltpu.VMEM_SHARED((NT, D), _f32))
    if with_deg:
        scratch.append(pltpu.VMEM_SHARED((NT, D), _f32))
    scratch.append(pltpu.SemaphoreType.DMA)
    return functools.partial(
        pl.kernel,
        out_type=out_type,
        mesh=_mesh(),
        compiler_params=pltpu.CompilerParams(use_tc_tiling_on_sc=False),
        scratch_types=scratch,
    )(_make_sc_scatter_body(with_deg))


def _sc_scatter(msg, dst, zer):
    return _sc_scatter_call(False)(msg, dst, zer)


def _sc_scatter_deg(msg, dst, zer, ones):
    return _sc_scatter_call(True)(msg, dst, zer, ones)


def kernel(x, edge_index, edge_attr, batch, W_emb, b_emb, We1, be1, We2, be2,
           W_root, b_conv, Wih, bih, Whh, bhh, Wih_l, bih_l, Whh_l, bhh_l):
    xp = jnp.pad(x, ((0, NT - N), (0, 0)))
    src = jnp.pad(edge_index[0], (0, EP - E)).reshape(NW, NCH, CH)
    dst = jnp.pad(edge_index[1], (0, EP - E),
                  constant_values=N).reshape(NW, NCH, CH)
    eap = jnp.pad(edge_attr, ((0, EP - E), (0, 0)))
    batchp = jnp.pad(batch, (0, NT - N), constant_values=B)
    oh = (batchp[:, None] == jnp.arange(B, dtype=batchp.dtype)).astype(_f32)
    zer = jnp.zeros((NT, D), _f32)
    ones = jnp.ones((CH, D), _f32)

    jj = jnp.arange(D * D)
    perm = (jj % D) * D + jj // D
    smat = (jj[:, None] // D == jnp.arange(D)[None, :]).astype(_f32)
    rmat = (jnp.arange(D)[:, None] == jj[None, :] % D).astype(_f32)

    nf = _prep(xp, W_emb.T, b_emb.reshape(1, D))
    eW = _edgemlp(eap, We1.T, be1.reshape(1, NF),
                  We2.T[:, perm].astype(jnp.bfloat16),
                  be2.reshape(1, D * D)[:, perm])

    st = nf
    degt = None
    for d in range(3):
        xj = _sc_gather(st, src)
        msg = _msg(xj, eW, rmat, smat)
        if d == 0:
            aggt, degt = _sc_scatter_deg(msg, dst, zer, ones)
        else:
            aggt = _sc_scatter(msg, dst, zer)
        st = _gru(aggt, degt, st, W_root.T, b_conv.reshape(1, D),
                  Wih.T, bih.reshape(1, 3 * D), Whh.T, bhh.reshape(1, 3 * D))

    return _s2s(st, oh, Wih_l.T, bih_l.reshape(1, 4 * D),
                Whh_l.T, bhh_l.reshape(1, 4 * D))


# speedup vs baseline: 3.9768x; 1.1234x over previous
import functools

import jax
import jax.numpy as jnp
from jax import lax
from jax.experimental import pallas as pl
from jax.experimental.pallas import tpu as pltpu
from jax.experimental.pallas import tpu_sc as plsc

N = 10000
NT = 10240
E = 160000
EP = 163840
D = 16
B = 64
NF = 128

NC = 2
NS = 16
NW = NC * NS
CH = 128
NCH = EP // (NW * CH)
TPT = EP // NW
RPT = NT // NS

BN = 2048
BE = 2000

_f32 = jnp.float32


@functools.lru_cache(None)
def _mesh():
    return plsc.VectorSubcoreMesh(
        core_axis_name="c", subcore_axis_name="s",
        num_cores=NC, num_subcores=NS)


def _prep_body(x_ref, w_ref, b_ref, nf_ref):
    nf_ref[...] = jax.nn.relu(
        jnp.dot(x_ref[...], w_ref[...], preferred_element_type=_f32)
        + b_ref[...])


def _prep(xp, WembT, b_emb2):
    return pl.pallas_call(
        _prep_body,
        grid=(NT // BN,),
        in_specs=[
            pl.BlockSpec((BN, NF), lambda i: (i, 0)),
            pl.BlockSpec((NF, D), lambda i: (0, 0)),
            pl.BlockSpec((1, D), lambda i: (0, 0)),
        ],
        out_specs=pl.BlockSpec((BN, D), lambda i: (i, 0)),
        out_shape=jax.ShapeDtypeStruct((NT, D), _f32),
    )(xp, WembT, b_emb2)


def _edgemlp_body(ea_ref, w1_ref, b1_ref, w2_ref, b2_ref, ew_ref):
    eh = jax.nn.relu(
        jnp.dot(ea_ref[...], w1_ref[...], preferred_element_type=_f32)
        + b1_ref[...])
    acc = jnp.dot(eh.astype(jnp.bfloat16), w2_ref[...],
                  preferred_element_type=_f32) + b2_ref[...]
    ew_ref[...] = acc.astype(jnp.bfloat16)


def _edgemlp(eap, We1T, be12, We2T, be22):
    return pl.pallas_call(
        _edgemlp_body,
        grid=(E // BE,),
        in_specs=[
            pl.BlockSpec((BE, D), lambda i: (i, 0)),
            pl.BlockSpec((D, NF), lambda i: (0, 0)),
            pl.BlockSpec((1, NF), lambda i: (0, 0)),
            pl.BlockSpec((NF, D * D), lambda i: (0, 0)),
            pl.BlockSpec((1, D * D), lambda i: (0, 0)),
        ],
        out_specs=pl.BlockSpec((BE, D * D), lambda i: (i, 0)),
        out_shape=jax.ShapeDtypeStruct((E, D * D), jnp.bfloat16),
    )(eap, We1T, be12, We2T, be22)


def _msg_body(xj_ref, ew_ref, r_ref, s_ref, msg_ref):
    xjr = jnp.dot(xj_ref[...], r_ref[...], preferred_element_type=_f32)
    msg_ref[...] = jnp.dot(xjr * ew_ref[...].astype(_f32), s_ref[...],
                           preferred_element_type=_f32)


def _msg(xj, eW, rmat, smat):
    return pl.pallas_call(
        _msg_body,
        grid=(E // BE,),
        in_specs=[
            pl.BlockSpec((BE, D), lambda i: (i, 0)),
            pl.BlockSpec((BE, D * D), lambda i: (i, 0)),
            pl.BlockSpec((D, D * D), lambda i: (0, 0)),
            pl.BlockSpec((D * D, D), lambda i: (0, 0)),
        ],
        out_specs=pl.BlockSpec((BE, D), lambda i: (i, 0)),
        out_shape=jax.ShapeDtypeStruct((E, D), _f32),
    )(xj, eW, rmat, smat)


def _gru_body(aggt_ref, degt_ref, st_ref, wr_ref, bc_ref, wih_ref, bih_ref,
              whh_ref, bhh_ref, out_ref):
    aggr = aggt_ref[0] + aggt_ref[1]
    deg = degt_ref[0][:, 0:1] + degt_ref[1][:, 0:1]
    invd = 1.0 / jnp.maximum(deg, 1.0)
    st = st_ref[...]
    conv = jax.nn.relu(
        aggr * invd
        + jnp.dot(st, wr_ref[...], preferred_element_type=_f32)
        + bc_ref[...])
    gi = jnp.dot(conv, wih_ref[...], preferred_element_type=_f32) + bih_ref[...]
    gh = jnp.dot(st, whh_ref[...], preferred_element_type=_f32) + bhh_ref[...]
    r = jax.nn.sigmoid(gi[:, 0:D] + gh[:, 0:D])
    z = jax.nn.sigmoid(gi[:, D:2 * D] + gh[:, D:2 * D])
    n = jnp.tanh(gi[:, 2 * D:3 * D] + r * gh[:, 2 * D:3 * D])
    out_ref[...] = (1.0 - z) * n + z * st


def _gru(aggt, degt, st, WrootT, bc2, WihT, bih2, WhhT, bhh2):
    return pl.pallas_call(
        _gru_body,
        grid=(NT // BN,),
        in_specs=[
            pl.BlockSpec((NC, BN, D), lambda i: (0, i, 0)),
            pl.BlockSpec((NC, BN, D), lambda i: (0, i, 0)),
            pl.BlockSpec((BN, D), lambda i: (i, 0)),
            pl.BlockSpec((D, D), lambda i: (0, 0)),
            pl.BlockSpec((1, D), lambda i: (0, 0)),
            pl.BlockSpec((D, 3 * D), lambda i: (0, 0)),
            pl.BlockSpec((1, 3 * D), lambda i: (0, 0)),
            pl.BlockSpec((D, 3 * D), lambda i: (0, 0)),
            pl.BlockSpec((1, 3 * D), lambda i: (0, 0)),
        ],
        out_specs=pl.BlockSpec((BN, D), lambda i: (i, 0)),
        out_shape=jax.ShapeDtypeStruct((NT, D), _f32),
    )(aggt, degt, st, WrootT, bc2, WihT, bih2, WhhT, bhh2)


def _s2s_body(nv_ref, oh_ref, wih_ref, bih_ref, whh_ref, bhh_ref, out_ref):
    nv = nv_ref[...]
    oh = oh_ref[...]
    q_star = jnp.zeros((B, 2 * D), _f32)
    hL = jnp.zeros((B, D), _f32)
    cL = jnp.zeros((B, D), _f32)
    for _ in range(3):
        gates = (jnp.dot(q_star, wih_ref[...], preferred_element_type=_f32)
                 + bih_ref[...]
                 + jnp.dot(hL, whh_ref[...], preferred_element_type=_f32)
                 + bhh_ref[...])
        i_ = jax.nn.sigmoid(gates[:, 0:D])
        f_ = jax.nn.sigmoid(gates[:, D:2 * D])
        g_ = jnp.tanh(gates[:, 2 * D:3 * D])
        o_ = jax.nn.sigmoid(gates[:, 3 * D:4 * D])
        cL = f_ * cL + i_ * g_
        hL = o_ * jnp.tanh(cL)
        qb = jnp.dot(oh, hL, preferred_element_type=_f32)
        e = jnp.sum(nv * qb, axis=1, keepdims=True)
        m = jnp.where(oh > 0.0, e, -jnp.inf)
        emax = jnp.max(m, axis=0, keepdims=True)
        emax = jnp.where(jnp.isfinite(emax), emax, 0.0)
        enode = jnp.dot(oh, emax.reshape(B, 1),
                        preferred_element_type=_f32)
        a = jnp.exp(e - enode)
        asum = lax.dot_general(oh, a, (((0,), (0,)), ((), ())),
                               preferred_element_type=_f32)
        anode = jnp.dot(oh, asum, preferred_element_type=_f32)
        a = a / (anode + 1e-16)
        r_ = lax.dot_general(oh, a * nv, (((0,), (0,)), ((), ())),
                             preferred_element_type=_f32)
        q_star = jnp.concatenate([hL, r_], axis=1)
    out_ref[...] = q_star


def _s2s(nv, oh, WihlT, bihl2, WhhlT, bhhl2):
    return pl.pallas_call(
        _s2s_body,
        out_shape=jax.ShapeDtypeStruct((B, 2 * D), _f32),
    )(nv, oh, WihlT, bihl2, WhhlT, bhhl2)


NCHL = (E - (NW - 1) * TPT) // CH


def _sc_gather_body(nv_hbm, ei_hbm, xj_hbm, idx_v, buf, sem):
    c = lax.axis_index("c")
    s = lax.axis_index("s")
    wid = c * NS + s
    base = wid * TPT
    nch = jnp.where(wid == NW - 1, NCHL, NCH)

    def stage(j, carry):
        pltpu.async_copy(ei_hbm.at[0].at[pl.ds(base + j * CH, CH)],
                         idx_v.at[j], sem)
        return carry

    lax.fori_loop(0, nch, stage, 0)

    def stage_drain(j, carry):
        pltpu.make_async_copy(ei_hbm.at[0].at[pl.ds(base + j * CH, CH)],
                              idx_v.at[j], sem).wait()
        return carry

    lax.fori_loop(0, nch, stage_drain, 0)

    def fire(j, carry):
        pltpu.async_copy(nv_hbm.at[idx_v.at[j]], buf.at[pl.ds(j * CH, CH)],
                         sem)
        return carry

    lax.fori_loop(0, nch, fire, 0)

    def drain(j, carry):
        pltpu.make_async_copy(nv_hbm.at[idx_v.at[j]],
                              buf.at[pl.ds(j * CH, CH)], sem).wait()
        pltpu.async_copy(buf.at[pl.ds(j * CH, CH)],
                         xj_hbm.at[pl.ds(base + j * CH, CH)], sem)
        return carry

    lax.fori_loop(0, nch, drain, 0)

    def out_drain(j, carry):
        pltpu.make_async_copy(buf.at[pl.ds(j * CH, CH)],
                              xj_hbm.at[pl.ds(base + j * CH, CH)], sem).wait()
        return carry

    lax.fori_loop(0, nch, out_drain, 0)


@functools.lru_cache(None)
def _sc_gather_call():
    return functools.partial(
        pl.kernel,
        out_type=jax.ShapeDtypeStruct((E, D), _f32),
        mesh=_mesh(),
        compiler_params=pltpu.CompilerParams(use_tc_tiling_on_sc=False),
        scratch_types=[
            pltpu.VMEM((NCH, CH), jnp.int32),
            pltpu.VMEM((TPT, D), _f32),
            pltpu.SemaphoreType.DMA,
        ],
    )(_sc_gather_body)


def _sc_gather(nv, ei):
    return _sc_gather_call()(nv, ei)


def _make_sc_scatter_body(with_deg):
    def body(*args):
        if with_deg:
            (msg_hbm, ei_hbm, zer_hbm, ones_hbm, out_hbm, deg_hbm,
             idx_v, mbuf, obuf, shared, shared_deg, sem) = args
        else:
            (msg_hbm, ei_hbm, zer_hbm, out_hbm,
             idx_v, mbuf, shared, sem) = args
        c = lax.axis_index("c")
        s = lax.axis_index("s")
        wid = c * NS + s
        base = wid * TPT
        nch = jnp.where(wid == NW - 1, NCHL, NCH)
        pltpu.sync_copy(zer_hbm.at[pl.ds(s * RPT, RPT)],
                        shared.at[pl.ds(s * RPT, RPT)])
        if with_deg:
            pltpu.sync_copy(zer_hbm.at[pl.ds(s * RPT, RPT)],
                            shared_deg.at[pl.ds(s * RPT, RPT)])
            pltpu.sync_copy(ones_hbm, obuf)

        def stage(j, carry):
            pltpu.async_copy(ei_hbm.at[1].at[pl.ds(base + j * CH, CH)],
                             idx_v.at[j], sem)
            pltpu.async_copy(msg_hbm.at[pl.ds(base + j * CH, CH)],
                             mbuf.at[pl.ds(j * CH, CH)], sem)
            return carry

        lax.fori_loop(0, nch, stage, 0)

        def stage_drain(j, carry):
            pltpu.make_async_copy(ei_hbm.at[1].at[pl.ds(base + j * CH, CH)],
                                  idx_v.at[j], sem).wait()
            pltpu.make_async_copy(msg_hbm.at[pl.ds(base + j * CH, CH)],
                                  mbuf.at[pl.ds(j * CH, CH)], sem).wait()
            return carry

        lax.fori_loop(0, nch, stage_drain, 0)
        plsc.subcore_barrier()

        def fire(j, carry):
            pltpu.async_copy(mbuf.at[pl.ds(j * CH, CH)],
                             shared.at[idx_v.at[j]], sem, add=True)
            if with_deg:
                pltpu.async_copy(obuf, shared_deg.at[idx_v.at[j]], sem,
                                 add=True)
            return carry

        lax.fori_loop(0, nch, fire, 0)

        def drain(j, carry):
            pltpu.make_async_copy(mbuf.at[pl.ds(j * CH, CH)],
                                  shared.at[idx_v.at[j]], sem).wait()
            if with_deg:
                pltpu.make_async_copy(obuf, shared_deg.at[idx_v.at[j]],
                                      sem).wait()
            return carry

        lax.fori_loop(0, nch, drain, 0)
        plsc.subcore_barrier()
        pltpu.sync_copy(shared.at[pl.ds(s * RPT, RPT)],
                        out_hbm.at[c].at[pl.ds(s * RPT, RPT)])
        if with_deg:
            pltpu.sync_copy(shared_deg.at[pl.ds(s * RPT, RPT)],
                            deg_hbm.at[c].at[pl.ds(s * RPT, RPT)])

    return body


@functools.lru_cache(None)
def _sc_scatter_call(with_deg):
    out_type = jax.ShapeDtypeStruct((NC, NT, D), _f32)
    scratch = [
        pltpu.VMEM((NCH, CH), jnp.int32),
        pltpu.VMEM((TPT, D), _f32),
    ]
    if with_deg:
        out_type = (out_type, jax.ShapeDtypeStruct((NC, NT, D), _f32))
        scratch.append(pltpu.VMEM((CH, D), _f32))
    scratch.append(pltpu.VMEM_SHARED((NT, D), _f32))
    if with_deg:
        scratch.append(pltpu.VMEM_SHARED((NT, D), _f32))
    scratch.append(pltpu.SemaphoreType.DMA)
    return functools.partial(
        pl.kernel,
        out_type=out_type,
        mesh=_mesh(),
        compiler_params=pltpu.CompilerParams(use_tc_tiling_on_sc=False),
        scratch_types=scratch,
    )(_make_sc_scatter_body(with_deg))


def _sc_scatter(msg, ei, zer):
    return _sc_scatter_call(False)(msg, ei, zer)


def _sc_scatter_deg(msg, ei, zer, ones):
    return _sc_scatter_call(True)(msg, ei, zer, ones)


def kernel(x, edge_index, edge_attr, batch, W_emb, b_emb, We1, be1, We2, be2,
           W_root, b_conv, Wih, bih, Whh, bhh, Wih_l, bih_l, Whh_l, bhh_l):
    xp = jnp.pad(x, ((0, NT - N), (0, 0)))
    batchp = jnp.pad(batch, (0, NT - N), constant_values=B)
    oh = (batchp[:, None] == jnp.arange(B, dtype=batchp.dtype)).astype(_f32)
    zer = jnp.zeros((NT, D), _f32)
    ones = jnp.ones((CH, D), _f32)

    jj = jnp.arange(D * D)
    perm = (jj % D) * D + jj // D
    smat = (jj[:, None] // D == jnp.arange(D)[None, :]).astype(_f32)
    rmat = (jnp.arange(D)[:, None] == jj[None, :] % D).astype(_f32)

    nf = _prep(xp, W_emb.T, b_emb.reshape(1, D))
    eW = _edgemlp(edge_attr, We1.T, be1.reshape(1, NF),
                  We2.T[:, perm].astype(jnp.bfloat16),
                  be2.reshape(1, D * D)[:, perm])

    st = nf
    degt = None
    for d in range(3):
        xj = _sc_gather(st, edge_index)
        msg = _msg(xj, eW, rmat, smat)
        if d == 0:
            aggt, degt = _sc_scatter_deg(msg, edge_index, zer, ones)
        else:
            aggt = _sc_scatter(msg, edge_index, zer)
        st = _gru(aggt, degt, st, W_root.T, b_conv.reshape(1, D),
                  Wih.T, bih.reshape(1, 3 * D), Whh.T, bhh.reshape(1, 3 * D))

    return _s2s(st, oh, Wih_l.T, bih_l.reshape(1, 4 * D),
                Whh_l.T, bhh_l.reshape(1, 4 * D))


# speedup vs baseline: 4.1536x; 1.0445x over previous
import functools

import jax
import jax.numpy as jnp
from jax import lax
from jax.experimental import pallas as pl
from jax.experimental.pallas import tpu as pltpu
from jax.experimental.pallas import tpu_sc as plsc

N = 10000
NT = 10240
E = 160000
EP = 163840
D = 16
B = 64
NF = 128

NC = 2
NS = 16
NW = NC * NS
CH = 128
NCH = EP // (NW * CH)
TPT = EP // NW
RPT = NT // NS

BN = 2048
BE = 2000

_f32 = jnp.float32


@functools.lru_cache(None)
def _mesh():
    return plsc.VectorSubcoreMesh(
        core_axis_name="c", subcore_axis_name="s",
        num_cores=NC, num_subcores=NS)


def _prep_body(x_ref, w_ref, b_ref, nf_ref):
    nf_ref[...] = jax.nn.relu(
        jnp.dot(x_ref[...], w_ref[...], preferred_element_type=_f32)
        + b_ref[...])


def _prep(xp, WembT, b_emb2):
    return pl.pallas_call(
        _prep_body,
        grid=(NT // BN,),
        in_specs=[
            pl.BlockSpec((BN, NF), lambda i: (i, 0)),
            pl.BlockSpec((NF, D), lambda i: (0, 0)),
            pl.BlockSpec((1, D), lambda i: (0, 0)),
        ],
        out_specs=pl.BlockSpec((BN, D), lambda i: (i, 0)),
        out_shape=jax.ShapeDtypeStruct((NT, D), _f32),
    )(xp, WembT, b_emb2)


def _msg_body(ea_ref, xj_ref, w1_ref, b1_ref, w2_ref, b2_ref, r_ref, s_ref,
              msg_ref):
    eh = jax.nn.relu(
        jnp.dot(ea_ref[...], w1_ref[...], preferred_element_type=_f32)
        + b1_ref[...])
    ew = jnp.dot(eh.astype(jnp.bfloat16), w2_ref[...],
                 preferred_element_type=_f32) + b2_ref[...]
    xjr = jnp.dot(xj_ref[...], r_ref[...], preferred_element_type=_f32)
    msg_ref[...] = jnp.dot(xjr * ew, s_ref[...], preferred_element_type=_f32)


def _msg(eap, xj, We1T, be12, We2T, be22, rmat, smat):
    return pl.pallas_call(
        _msg_body,
        grid=(E // BE,),
        in_specs=[
            pl.BlockSpec((BE, D), lambda i: (i, 0)),
            pl.BlockSpec((BE, D), lambda i: (i, 0)),
            pl.BlockSpec((D, NF), lambda i: (0, 0)),
            pl.BlockSpec((1, NF), lambda i: (0, 0)),
            pl.BlockSpec((NF, D * D), lambda i: (0, 0)),
            pl.BlockSpec((1, D * D), lambda i: (0, 0)),
            pl.BlockSpec((D, D * D), lambda i: (0, 0)),
            pl.BlockSpec((D * D, D), lambda i: (0, 0)),
        ],
        out_specs=pl.BlockSpec((BE, D), lambda i: (i, 0)),
        out_shape=jax.ShapeDtypeStruct((E, D), _f32),
    )(eap, xj, We1T, be12, We2T, be22, rmat, smat)


def _gru_body(aggt_ref, degt_ref, st_ref, wr_ref, bc_ref, wih_ref, bih_ref,
              whh_ref, bhh_ref, out_ref):
    aggr = aggt_ref[0] + aggt_ref[1]
    deg = degt_ref[0][:, 0:1] + degt_ref[1][:, 0:1]
    invd = 1.0 / jnp.maximum(deg, 1.0)
    st = st_ref[...]
    conv = jax.nn.relu(
        aggr * invd
        + jnp.dot(st, wr_ref[...], preferred_element_type=_f32)
        + bc_ref[...])
    gi = jnp.dot(conv, wih_ref[...], preferred_element_type=_f32) + bih_ref[...]
    gh = jnp.dot(st, whh_ref[...], preferred_element_type=_f32) + bhh_ref[...]
    r = jax.nn.sigmoid(gi[:, 0:D] + gh[:, 0:D])
    z = jax.nn.sigmoid(gi[:, D:2 * D] + gh[:, D:2 * D])
    n = jnp.tanh(gi[:, 2 * D:3 * D] + r * gh[:, 2 * D:3 * D])
    out_ref[...] = (1.0 - z) * n + z * st


def _gru(aggt, degt, st, WrootT, bc2, WihT, bih2, WhhT, bhh2):
    return pl.pallas_call(
        _gru_body,
        grid=(NT // BN,),
        in_specs=[
            pl.BlockSpec((NC, BN, D), lambda i: (0, i, 0)),
            pl.BlockSpec((NC, BN, D), lambda i: (0, i, 0)),
            pl.BlockSpec((BN, D), lambda i: (i, 0)),
            pl.BlockSpec((D, D), lambda i: (0, 0)),
            pl.BlockSpec((1, D), lambda i: (0, 0)),
            pl.BlockSpec((D, 3 * D), lambda i: (0, 0)),
            pl.BlockSpec((1, 3 * D), lambda i: (0, 0)),
            pl.BlockSpec((D, 3 * D), lambda i: (0, 0)),
            pl.BlockSpec((1, 3 * D), lambda i: (0, 0)),
        ],
        out_specs=pl.BlockSpec((BN, D), lambda i: (i, 0)),
        out_shape=jax.ShapeDtypeStruct((NT, D), _f32),
    )(aggt, degt, st, WrootT, bc2, WihT, bih2, WhhT, bhh2)


def _s2s_body(nv_ref, oh_ref, wih_ref, bih_ref, whh_ref, bhh_ref, out_ref):
    nv = nv_ref[...]
    oh = oh_ref[...]
    q_star = jnp.zeros((B, 2 * D), _f32)
    hL = jnp.zeros((B, D), _f32)
    cL = jnp.zeros((B, D), _f32)
    for _ in range(3):
        gates = (jnp.dot(q_star, wih_ref[...], preferred_element_type=_f32)
                 + bih_ref[...]
                 + jnp.dot(hL, whh_ref[...], preferred_element_type=_f32)
                 + bhh_ref[...])
        i_ = jax.nn.sigmoid(gates[:, 0:D])
        f_ = jax.nn.sigmoid(gates[:, D:2 * D])
        g_ = jnp.tanh(gates[:, 2 * D:3 * D])
        o_ = jax.nn.sigmoid(gates[:, 3 * D:4 * D])
        cL = f_ * cL + i_ * g_
        hL = o_ * jnp.tanh(cL)
        qb = jnp.dot(oh, hL, preferred_element_type=_f32)
        e = jnp.sum(nv * qb, axis=1, keepdims=True)
        m = jnp.where(oh > 0.0, e, -jnp.inf)
        emax = jnp.max(m, axis=0, keepdims=True)
        emax = jnp.where(jnp.isfinite(emax), emax, 0.0)
        enode = jnp.dot(oh, emax.reshape(B, 1),
                        preferred_element_type=_f32)
        a = jnp.exp(e - enode)
        asum = lax.dot_general(oh, a, (((0,), (0,)), ((), ())),
                               preferred_element_type=_f32)
        anode = jnp.dot(oh, asum, preferred_element_type=_f32)
        a = a / (anode + 1e-16)
        r_ = lax.dot_general(oh, a * nv, (((0,), (0,)), ((), ())),
                             preferred_element_type=_f32)
        q_star = jnp.concatenate([hL, r_], axis=1)
    out_ref[...] = q_star


def _s2s(nv, oh, WihlT, bihl2, WhhlT, bhhl2):
    return pl.pallas_call(
        _s2s_body,
        out_shape=jax.ShapeDtypeStruct((B, 2 * D), _f32),
    )(nv, oh, WihlT, bihl2, WhhlT, bhhl2)


NCHL = (E - (NW - 1) * TPT) // CH


def _sc_gather_body(nv_hbm, ei_hbm, xj_hbm, idx_v, buf, sem):
    c = lax.axis_index("c")
    s = lax.axis_index("s")
    wid = c * NS + s
    base = wid * TPT
    nch = jnp.where(wid == NW - 1, NCHL, NCH)

    def stage(j, carry):
        pltpu.async_copy(ei_hbm.at[0].at[pl.ds(base + j * CH, CH)],
                         idx_v.at[j], sem)
        return carry

    lax.fori_loop(0, nch, stage, 0)

    def stage_drain(j, carry):
        pltpu.make_async_copy(ei_hbm.at[0].at[pl.ds(base + j * CH, CH)],
                              idx_v.at[j], sem).wait()
        return carry

    lax.fori_loop(0, nch, stage_drain, 0)

    def fire(j, carry):
        pltpu.async_copy(nv_hbm.at[idx_v.at[j]], buf.at[pl.ds(j * CH, CH)],
                         sem)
        return carry

    lax.fori_loop(0, nch, fire, 0)

    def drain(j, carry):
        pltpu.make_async_copy(nv_hbm.at[idx_v.at[j]],
                              buf.at[pl.ds(j * CH, CH)], sem).wait()
        pltpu.async_copy(buf.at[pl.ds(j * CH, CH)],
                         xj_hbm.at[pl.ds(base + j * CH, CH)], sem)
        return carry

    lax.fori_loop(0, nch, drain, 0)

    def out_drain(j, carry):
        pltpu.make_async_copy(buf.at[pl.ds(j * CH, CH)],
                              xj_hbm.at[pl.ds(base + j * CH, CH)], sem).wait()
        return carry

    lax.fori_loop(0, nch, out_drain, 0)


@functools.lru_cache(None)
def _sc_gather_call():
    return functools.partial(
        pl.kernel,
        out_type=jax.ShapeDtypeStruct((E, D), _f32),
        mesh=_mesh(),
        compiler_params=pltpu.CompilerParams(use_tc_tiling_on_sc=False),
        scratch_types=[
            pltpu.VMEM((NCH, CH), jnp.int32),
            pltpu.VMEM((TPT, D), _f32),
            pltpu.SemaphoreType.DMA,
        ],
    )(_sc_gather_body)


def _sc_gather(nv, ei):
    return _sc_gather_call()(nv, ei)


def _make_sc_scatter_body(with_deg):
    def body(*args):
        if with_deg:
            (msg_hbm, ei_hbm, zer_hbm, ones_hbm, out_hbm, deg_hbm,
             idx_v, mbuf, obuf, shared, shared_deg, sem) = args
        else:
            (msg_hbm, ei_hbm, zer_hbm, out_hbm,
             idx_v, mbuf, shared, sem) = args
        c = lax.axis_index("c")
        s = lax.axis_index("s")
        wid = c * NS + s
        base = wid * TPT
        nch = jnp.where(wid == NW - 1, NCHL, NCH)
        pltpu.sync_copy(zer_hbm.at[pl.ds(s * RPT, RPT)],
                        shared.at[pl.ds(s * RPT, RPT)])
        if with_deg:
            pltpu.sync_copy(zer_hbm.at[pl.ds(s * RPT, RPT)],
                            shared_deg.at[pl.ds(s * RPT, RPT)])
            pltpu.sync_copy(ones_hbm, obuf)

        def stage(j, carry):
            pltpu.async_copy(ei_hbm.at[1].at[pl.ds(base + j * CH, CH)],
                             idx_v.at[j], sem)
            pltpu.async_copy(msg_hbm.at[pl.ds(base + j * CH, CH)],
                             mbuf.at[pl.ds(j * CH, CH)], sem)
            return carry

        lax.fori_loop(0, nch, stage, 0)

        def stage_drain(j, carry):
            pltpu.make_async_copy(ei_hbm.at[1].at[pl.ds(base + j * CH, CH)],
                                  idx_v.at[j], sem).wait()
            pltpu.make_async_copy(msg_hbm.at[pl.ds(base + j * CH, CH)],
                                  mbuf.at[pl.ds(j * CH, CH)], sem).wait()
            return carry

        lax.fori_loop(0, nch, stage_drain, 0)
        plsc.subcore_barrier()

        def fire(j, carry):
            pltpu.async_copy(mbuf.at[pl.ds(j * CH, CH)],
                             shared.at[idx_v.at[j]], sem, add=True)
            if with_deg:
                pltpu.async_copy(obuf, shared_deg.at[idx_v.at[j]], sem,
                                 add=True)
            return carry

        lax.fori_loop(0, nch, fire, 0)

        def drain(j, carry):
            pltpu.make_async_copy(mbuf.at[pl.ds(j * CH, CH)],
                                  shared.at[idx_v.at[j]], sem).wait()
            if with_deg:
                pltpu.make_async_copy(obuf, shared_deg.at[idx_v.at[j]],
                                      sem).wait()
            return carry

        lax.fori_loop(0, nch, drain, 0)
        plsc.subcore_barrier()
        pltpu.sync_copy(shared.at[pl.ds(s * RPT, RPT)],
                        out_hbm.at[c].at[pl.ds(s * RPT, RPT)])
        if with_deg:
            pltpu.sync_copy(shared_deg.at[pl.ds(s * RPT, RPT)],
                            deg_hbm.at[c].at[pl.ds(s * RPT, RPT)])

    return body


@functools.lru_cache(None)
def _sc_scatter_call(with_deg):
    out_type = jax.ShapeDtypeStruct((NC, NT, D), _f32)
    scratch = [
        pltpu.VMEM((NCH, CH), jnp.int32),
        pltpu.VMEM((TPT, D), _f32),
    ]
    if with_deg:
        out_type = (out_type, jax.ShapeDtypeStruct((NC, NT, D), _f32))
        scratch.append(pltpu.VMEM((CH, D), _f32))
    scratch.append(pltpu.VMEM_SHARED((NT, D), _f32))
    if with_deg:
        scratch.append(pltpu.VMEM_SHARED((NT, D), _f32))
    scratch.append(pltpu.SemaphoreType.DMA)
    return functools.partial(
        pl.kernel,
        out_type=out_type,
        mesh=_mesh(),
        compiler_params=pltpu.CompilerParams(use_tc_tiling_on_sc=False),
        scratch_types=scratch,
    )(_make_sc_scatter_body(with_deg))


def _sc_scatter(msg, ei, zer):
    return _sc_scatter_call(False)(msg, ei, zer)


def _sc_scatter_deg(msg, ei, zer, ones):
    return _sc_scatter_call(True)(msg, ei, zer, ones)


def kernel(x, edge_index, edge_attr, batch, W_emb, b_emb, We1, be1, We2, be2,
           W_root, b_conv, Wih, bih, Whh, bhh, Wih_l, bih_l, Whh_l, bhh_l):
    xp = jnp.pad(x, ((0, NT - N), (0, 0)))
    batchp = jnp.pad(batch, (0, NT - N), constant_values=B)
    oh = (batchp[:, None] == jnp.arange(B, dtype=batchp.dtype)).astype(_f32)
    zer = jnp.zeros((NT, D), _f32)
    ones = jnp.ones((CH, D), _f32)

    jj = jnp.arange(D * D)
    perm = (jj % D) * D + jj // D
    smat = (jj[:, None] // D == jnp.arange(D)[None, :]).astype(_f32)
    rmat = (jnp.arange(D)[:, None] == jj[None, :] % D).astype(_f32)

    nf = _prep(xp, W_emb.T, b_emb.reshape(1, D))
    We2pT = We2.T[:, perm].astype(jnp.bfloat16)
    be2p = be2.reshape(1, D * D)[:, perm]
    be12 = be1.reshape(1, NF)
    We1T = We1.T

    st = nf
    degt = None
    for d in range(3):
        xj = _sc_gather(st, edge_index)
        msg = _msg(edge_attr, xj, We1T, be12, We2pT, be2p, rmat, smat)
        if d == 0:
            aggt, degt = _sc_scatter_deg(msg, edge_index, zer, ones)
        else:
            aggt = _sc_scatter(msg, edge_index, zer)
        st = _gru(aggt, degt, st, W_root.T, b_conv.reshape(1, D),
                  Wih.T, bih.reshape(1, 3 * D), Whh.T, bhh.reshape(1, 3 * D))

    return _s2s(st, oh, Wih_l.T, bih_l.reshape(1, 4 * D),
                Whh_l.T, bhh_l.reshape(1, 4 * D))


# speedup vs baseline: 4.6746x; 1.1254x over previous
import functools

import jax
import jax.numpy as jnp
from jax import lax
from jax.experimental import pallas as pl
from jax.experimental.pallas import tpu as pltpu
from jax.experimental.pallas import tpu_sc as plsc

N = 10000
NT = 10240
E = 160000
EP = 163840
D = 16
B = 64
NF = 128

NC = 2
NS = 16
NW = NC * NS
CH = 128
NCH = EP // (NW * CH)
TPT = EP // NW
RPT = NT // NS

BN = 10240
BE = 8000

_f32 = jnp.float32


@functools.lru_cache(None)
def _mesh():
    return plsc.VectorSubcoreMesh(
        core_axis_name="c", subcore_axis_name="s",
        num_cores=NC, num_subcores=NS)


def _prep_body(x_ref, w_ref, b_ref, nf_ref):
    nf_ref[...] = jax.nn.relu(
        jnp.dot(x_ref[...], w_ref[...], preferred_element_type=_f32)
        + b_ref[...])


def _prep(xp, WembT, b_emb2):
    return pl.pallas_call(
        _prep_body,
        grid=(NT // BN,),
        in_specs=[
            pl.BlockSpec((BN, NF), lambda i: (i, 0)),
            pl.BlockSpec((NF, D), lambda i: (0, 0)),
            pl.BlockSpec((1, D), lambda i: (0, 0)),
        ],
        out_specs=pl.BlockSpec((BN, D), lambda i: (i, 0)),
        out_shape=jax.ShapeDtypeStruct((NT, D), _f32),
    )(xp, WembT, b_emb2)


def _msg_body(ea_ref, xj_ref, w1_ref, b1_ref, w2_ref, b2_ref, r_ref, s_ref,
              msg_ref):
    eh = jax.nn.relu(
        jnp.dot(ea_ref[...], w1_ref[...], preferred_element_type=_f32)
        + b1_ref[...])
    ew = jnp.dot(eh.astype(jnp.bfloat16), w2_ref[...],
                 preferred_element_type=_f32) + b2_ref[...]
    xjr = jnp.dot(xj_ref[...], r_ref[...], preferred_element_type=_f32)
    msg_ref[...] = jnp.dot(xjr * ew, s_ref[...], preferred_element_type=_f32)


def _msg(eap, xj, We1T, be12, We2T, be22, rmat, smat):
    return pl.pallas_call(
        _msg_body,
        grid=(E // BE,),
        in_specs=[
            pl.BlockSpec((BE, D), lambda i: (i, 0)),
            pl.BlockSpec((BE, D), lambda i: (i, 0)),
            pl.BlockSpec((D, NF), lambda i: (0, 0)),
            pl.BlockSpec((1, NF), lambda i: (0, 0)),
            pl.BlockSpec((NF, D * D), lambda i: (0, 0)),
            pl.BlockSpec((1, D * D), lambda i: (0, 0)),
            pl.BlockSpec((D, D * D), lambda i: (0, 0)),
            pl.BlockSpec((D * D, D), lambda i: (0, 0)),
        ],
        out_specs=pl.BlockSpec((BE, D), lambda i: (i, 0)),
        out_shape=jax.ShapeDtypeStruct((E, D), _f32),
    )(eap, xj, We1T, be12, We2T, be22, rmat, smat)


def _gru_body(aggt_ref, degt_ref, st_ref, wr_ref, bc_ref, wih_ref, bih_ref,
              whh_ref, bhh_ref, out_ref):
    aggr = aggt_ref[0] + aggt_ref[1]
    deg = degt_ref[0][:, 0:1] + degt_ref[1][:, 0:1]
    invd = 1.0 / jnp.maximum(deg, 1.0)
    st = st_ref[...]
    conv = jax.nn.relu(
        aggr * invd
        + jnp.dot(st, wr_ref[...], preferred_element_type=_f32)
        + bc_ref[...])
    gi = jnp.dot(conv, wih_ref[...], preferred_element_type=_f32) + bih_ref[...]
    gh = jnp.dot(st, whh_ref[...], preferred_element_type=_f32) + bhh_ref[...]
    r = jax.nn.sigmoid(gi[:, 0:D] + gh[:, 0:D])
    z = jax.nn.sigmoid(gi[:, D:2 * D] + gh[:, D:2 * D])
    n = jnp.tanh(gi[:, 2 * D:3 * D] + r * gh[:, 2 * D:3 * D])
    out_ref[...] = (1.0 - z) * n + z * st


def _gru(aggt, degt, st, WrootT, bc2, WihT, bih2, WhhT, bhh2):
    return pl.pallas_call(
        _gru_body,
        grid=(NT // BN,),
        in_specs=[
            pl.BlockSpec((NC, BN, D), lambda i: (0, i, 0)),
            pl.BlockSpec((NC, BN, D), lambda i: (0, i, 0)),
            pl.BlockSpec((BN, D), lambda i: (i, 0)),
            pl.BlockSpec((D, D), lambda i: (0, 0)),
            pl.BlockSpec((1, D), lambda i: (0, 0)),
            pl.BlockSpec((D, 3 * D), lambda i: (0, 0)),
            pl.BlockSpec((1, 3 * D), lambda i: (0, 0)),
            pl.BlockSpec((D, 3 * D), lambda i: (0, 0)),
            pl.BlockSpec((1, 3 * D), lambda i: (0, 0)),
        ],
        out_specs=pl.BlockSpec((BN, D), lambda i: (i, 0)),
        out_shape=jax.ShapeDtypeStruct((NT, D), _f32),
    )(aggt, degt, st, WrootT, bc2, WihT, bih2, WhhT, bhh2)


def _s2s_body(nv_ref, oh_ref, wih_ref, bih_ref, whh_ref, bhh_ref, out_ref):
    nv = nv_ref[...]
    oh = oh_ref[...]
    q_star = jnp.zeros((B, 2 * D), _f32)
    hL = jnp.zeros((B, D), _f32)
    cL = jnp.zeros((B, D), _f32)
    for _ in range(3):
        gates = (jnp.dot(q_star, wih_ref[...], preferred_element_type=_f32)
                 + bih_ref[...]
                 + jnp.dot(hL, whh_ref[...], preferred_element_type=_f32)
                 + bhh_ref[...])
        i_ = jax.nn.sigmoid(gates[:, 0:D])
        f_ = jax.nn.sigmoid(gates[:, D:2 * D])
        g_ = jnp.tanh(gates[:, 2 * D:3 * D])
        o_ = jax.nn.sigmoid(gates[:, 3 * D:4 * D])
        cL = f_ * cL + i_ * g_
        hL = o_ * jnp.tanh(cL)
        qb = jnp.dot(oh, hL, preferred_element_type=_f32)
        e = jnp.sum(nv * qb, axis=1, keepdims=True)
        m = jnp.where(oh > 0.0, e, -jnp.inf)
        emax = jnp.max(m, axis=0, keepdims=True)
        emax = jnp.where(jnp.isfinite(emax), emax, 0.0)
        enode = jnp.dot(oh, emax.reshape(B, 1),
                        preferred_element_type=_f32)
        a = jnp.exp(e - enode)
        asum = lax.dot_general(oh, a, (((0,), (0,)), ((), ())),
                               preferred_element_type=_f32)
        anode = jnp.dot(oh, asum, preferred_element_type=_f32)
        a = a / (anode + 1e-16)
        r_ = lax.dot_general(oh, a * nv, (((0,), (0,)), ((), ())),
                             preferred_element_type=_f32)
        q_star = jnp.concatenate([hL, r_], axis=1)
    out_ref[...] = q_star


def _s2s(nv, oh, WihlT, bihl2, WhhlT, bhhl2):
    return pl.pallas_call(
        _s2s_body,
        out_shape=jax.ShapeDtypeStruct((B, 2 * D), _f32),
    )(nv, oh, WihlT, bihl2, WhhlT, bhhl2)


NCHL = (E - (NW - 1) * TPT) // CH


def _sc_gather_body(nv_hbm, ei_hbm, xj_hbm, idx_v, buf, sem):
    c = lax.axis_index("c")
    s = lax.axis_index("s")
    wid = c * NS + s
    base = wid * TPT
    nch = jnp.where(wid == NW - 1, NCHL, NCH)

    def stage(j, carry):
        pltpu.async_copy(ei_hbm.at[0].at[pl.ds(base + j * CH, CH)],
                         idx_v.at[j], sem)
        return carry

    lax.fori_loop(0, nch, stage, 0)

    def stage_drain(j, carry):
        pltpu.make_async_copy(ei_hbm.at[0].at[pl.ds(base + j * CH, CH)],
                              idx_v.at[j], sem).wait()
        return carry

    lax.fori_loop(0, nch, stage_drain, 0)

    def fire(j, carry):
        pltpu.async_copy(nv_hbm.at[idx_v.at[j]], buf.at[pl.ds(j * CH, CH)],
                         sem)
        return carry

    lax.fori_loop(0, nch, fire, 0)

    def drain(j, carry):
        pltpu.make_async_copy(nv_hbm.at[idx_v.at[j]],
                              buf.at[pl.ds(j * CH, CH)], sem).wait()
        pltpu.async_copy(buf.at[pl.ds(j * CH, CH)],
                         xj_hbm.at[pl.ds(base + j * CH, CH)], sem)
        return carry

    lax.fori_loop(0, nch, drain, 0)

    def out_drain(j, carry):
        pltpu.make_async_copy(buf.at[pl.ds(j * CH, CH)],
                              xj_hbm.at[pl.ds(base + j * CH, CH)], sem).wait()
        return carry

    lax.fori_loop(0, nch, out_drain, 0)


@functools.lru_cache(None)
def _sc_gather_call():
    return functools.partial(
        pl.kernel,
        out_type=jax.ShapeDtypeStruct((E, D), _f32),
        mesh=_mesh(),
        compiler_params=pltpu.CompilerParams(use_tc_tiling_on_sc=False),
        scratch_types=[
            pltpu.VMEM((NCH, CH), jnp.int32),
            pltpu.VMEM((TPT, D), _f32),
            pltpu.SemaphoreType.DMA,
        ],
    )(_sc_gather_body)


def _sc_gather(nv, ei):
    return _sc_gather_call()(nv, ei)


def _make_sc_scatter_body(with_deg):
    def body(*args):
        if with_deg:
            (msg_hbm, ei_hbm, zer_hbm, ones_hbm, out_hbm, deg_hbm,
             idx_v, mbuf, obuf, shared, shared_deg, sem) = args
        else:
            (msg_hbm, ei_hbm, zer_hbm, out_hbm,
             idx_v, mbuf, shared, sem) = args
        c = lax.axis_index("c")
        s = lax.axis_index("s")
        wid = c * NS + s
        base = wid * TPT
        nch = jnp.where(wid == NW - 1, NCHL, NCH)
        pltpu.sync_copy(zer_hbm.at[pl.ds(s * RPT, RPT)],
                        shared.at[pl.ds(s * RPT, RPT)])
        if with_deg:
            pltpu.sync_copy(zer_hbm.at[pl.ds(s * RPT, RPT)],
                            shared_deg.at[pl.ds(s * RPT, RPT)])
            pltpu.sync_copy(ones_hbm, obuf)

        def stage(j, carry):
            pltpu.async_copy(ei_hbm.at[1].at[pl.ds(base + j * CH, CH)],
                             idx_v.at[j], sem)
            pltpu.async_copy(msg_hbm.at[pl.ds(base + j * CH, CH)],
                             mbuf.at[pl.ds(j * CH, CH)], sem)
            return carry

        lax.fori_loop(0, nch, stage, 0)

        def stage_drain(j, carry):
            pltpu.make_async_copy(ei_hbm.at[1].at[pl.ds(base + j * CH, CH)],
                                  idx_v.at[j], sem).wait()
            pltpu.make_async_copy(msg_hbm.at[pl.ds(base + j * CH, CH)],
                                  mbuf.at[pl.ds(j * CH, CH)], sem).wait()
            return carry

        lax.fori_loop(0, nch, stage_drain, 0)
        plsc.subcore_barrier()

        def fire(j, carry):
            pltpu.async_copy(mbuf.at[pl.ds(j * CH, CH)],
                             shared.at[idx_v.at[j]], sem, add=True)
            if with_deg:
                pltpu.async_copy(obuf, shared_deg.at[idx_v.at[j]], sem,
                                 add=True)
            return carry

        lax.fori_loop(0, nch, fire, 0)

        def drain(j, carry):
            pltpu.make_async_copy(mbuf.at[pl.ds(j * CH, CH)],
                                  shared.at[idx_v.at[j]], sem).wait()
            if with_deg:
                pltpu.make_async_copy(obuf, shared_deg.at[idx_v.at[j]],
                                      sem).wait()
            return carry

        lax.fori_loop(0, nch, drain, 0)
        plsc.subcore_barrier()
        pltpu.sync_copy(shared.at[pl.ds(s * RPT, RPT)],
                        out_hbm.at[c].at[pl.ds(s * RPT, RPT)])
        if with_deg:
            pltpu.sync_copy(shared_deg.at[pl.ds(s * RPT, RPT)],
                            deg_hbm.at[c].at[pl.ds(s * RPT, RPT)])

    return body


@functools.lru_cache(None)
def _sc_scatter_call(with_deg):
    out_type = jax.ShapeDtypeStruct((NC, NT, D), _f32)
    scratch = [
        pltpu.VMEM((NCH, CH), jnp.int32),
        pltpu.VMEM((TPT, D), _f32),
    ]
    if with_deg:
        out_type = (out_type, jax.ShapeDtypeStruct((NC, NT, D), _f32))
        scratch.append(pltpu.VMEM((CH, D), _f32))
    scratch.append(pltpu.VMEM_SHARED((NT, D), _f32))
    if with_deg:
        scratch.append(pltpu.VMEM_SHARED((NT, D), _f32))
    scratch.append(pltpu.SemaphoreType.DMA)
    return functools.partial(
        pl.kernel,
        out_type=out_type,
        mesh=_mesh(),
        compiler_params=pltpu.CompilerParams(use_tc_tiling_on_sc=False),
        scratch_types=scratch,
    )(_make_sc_scatter_body(with_deg))


def _sc_scatter(msg, ei, zer):
    return _sc_scatter_call(False)(msg, ei, zer)


def _sc_scatter_deg(msg, ei, zer, ones):
    return _sc_scatter_call(True)(msg, ei, zer, ones)


def kernel(x, edge_index, edge_attr, batch, W_emb, b_emb, We1, be1, We2, be2,
           W_root, b_conv, Wih, bih, Whh, bhh, Wih_l, bih_l, Whh_l, bhh_l):
    xp = jnp.pad(x, ((0, NT - N), (0, 0)))
    batchp = jnp.pad(batch, (0, NT - N), constant_values=B)
    oh = (batchp[:, None] == jnp.arange(B, dtype=batchp.dtype)).astype(_f32)
    zer = jnp.zeros((NT, D), _f32)
    ones = jnp.ones((CH, D), _f32)

    jj = jnp.arange(D * D)
    perm = (jj % D) * D + jj // D
    smat = (jj[:, None] // D == jnp.arange(D)[None, :]).astype(_f32)
    rmat = (jnp.arange(D)[:, None] == jj[None, :] % D).astype(_f32)

    nf = _prep(xp, W_emb.T, b_emb.reshape(1, D))
    We2pT = We2.T[:, perm].astype(jnp.bfloat16)
    be2p = be2.reshape(1, D * D)[:, perm]
    be12 = be1.reshape(1, NF)
    We1T = We1.T

    st = nf
    degt = None
    for d in range(3):
        xj = _sc_gather(st, edge_index)
        msg = _msg(edge_attr, xj, We1T, be12, We2pT, be2p, rmat, smat)
        if d == 0:
            aggt, degt = _sc_scatter_deg(msg, edge_index, zer, ones)
        else:
            aggt = _sc_scatter(msg, edge_index, zer)
        st = _gru(aggt, degt, st, W_root.T, b_conv.reshape(1, D),
                  Wih.T, bih.reshape(1, 3 * D), Whh.T, bhh.reshape(1, 3 * D))

    return _s2s(st, oh, Wih_l.T, bih_l.reshape(1, 4 * D),
                Whh_l.T, bhh_l.reshape(1, 4 * D))
